# baseline (speedup 1.0000x reference)
import functools

import jax
import jax.numpy as jnp
import numpy as np
from jax import lax
from jax.experimental import pallas as pl
from jax.experimental.pallas import tpu as pltpu
from jax.experimental.pallas import tpu_sc as plsc

TEMPERATURE = 0.07
N_NEGATIVES = 32
_B = 64
_D = 64
_HW = 128 * 128
_N2 = 2 * _B


def _pool_body(f_ref, l_ref, t_ref, b_ref, c_ref):
    f = f_ref[0]
    l0 = l_ref[0, 0]
    l1 = l_ref[0, 1]
    t0p = jnp.sum(f * l0[None, :, :], axis=1)
    t1p = jnp.sum(f * l1[None, :, :], axis=1)
    sp = jnp.sum(f, axis=1)
    t0 = jnp.sum(t0p, axis=1)
    t1 = jnp.sum(t1p, axis=1)
    s = jnp.sum(sp, axis=1)
    t_ref[0, 0] = t0
    t_ref[0, 1] = t1
    b_ref[0, 0] = s - t0
    b_ref[0, 1] = s - t1
    c_ref[0, 0] = jnp.broadcast_to(jnp.sum(l0), (_D,))
    c_ref[0, 1] = jnp.broadcast_to(jnp.sum(l1), (_D,))


_NC = 2
_NS = 16
_DT = 16


def _sc_pool_body(f_hbm, l_hbm, out_hbm, lab_v, fbuf, out_v, sem0, sem1):
    wid = lax.axis_index("s") * _NC + lax.axis_index("c")

    def do_image(img, carry):
        b = wid * 2 + img
        pltpu.sync_copy(l_hbm.at[b], lab_v)

        def cnt_body(i, cc):
            c0, c1 = cc
            h = i >> 3
            w0 = (i & 7) * 16
            return (c0 + lab_v[0, h, pl.ds(w0, 16)],
                    c1 + lab_v[1, h, pl.ds(w0, 16)])

        c0v, c1v = lax.fori_loop(
            0, 1024, cnt_body,
            (jnp.zeros((16,), jnp.float32), jnp.zeros((16,), jnp.float32)))
        out_v[pl.ds(4 * _D * 16, 16)] = c0v
        out_v[pl.ds(5 * _D * 16, 16)] = c1v

        sems = (sem0, sem1)
        for dt in range(_D // _DT):
            copies = [None, None]
            copies[0] = pltpu.async_copy(
                f_hbm.at[b, pl.ds(dt * _DT, _DT), pl.ds(0, 16), :],
                fbuf.at[0], sem0)
            accs = tuple(jnp.zeros((16,), jnp.float32) for _ in range(3 * _DT))
            for hc in range(8):
                cur = hc % 2
                if hc < 7:
                    nxt = (hc + 1) % 2
                    copies[nxt] = pltpu.async_copy(
                        f_hbm.at[b, pl.ds(dt * _DT, _DT),
                                 pl.ds((hc + 1) * 16, 16), :],
                        fbuf.at[nxt], sems[nxt])
                copies[cur].wait()

                def chunk_body(i, acc, _cur=cur, _hc=hc):
                    h = i >> 3
                    w0 = (i & 7) * 16
                    l0c = lab_v[0, _hc * 16 + h, pl.ds(w0, 16)]
                    l1c = lab_v[1, _hc * 16 + h, pl.ds(w0, 16)]
                    t0s, t1s, ss = [], [], []
                    for j in range(_DT):
                        fv = fbuf[_cur, j, h, pl.ds(w0, 16)]
                        t0s.append(acc[j] + fv * l0c)
                        t1s.append(acc[_DT + j] + fv * l1c)
                        ss.append(acc[2 * _DT + j] + fv)
                    return tuple(t0s + t1s + ss)

                accs = lax.fori_loop(0, 128, chunk_body, accs)
            for j in range(_DT):
                d = dt * _DT + j
                out_v[pl.ds((0 * _D + d) * 16, 16)] = accs[j]
                out_v[pl.ds((1 * _D + d) * 16, 16)] = accs[_DT + j]
                out_v[pl.ds((2 * _D + d) * 16, 16)] = \
                    accs[2 * _DT + j] - accs[j]
                out_v[pl.ds((3 * _D + d) * 16, 16)] = \
                    accs[2 * _DT + j] - accs[_DT + j]
        pltpu.sync_copy(out_v, out_hbm.at[b])
        return carry

    lax.fori_loop(0, 2, do_image, 0)


_sc_pool = functools.partial(
    pl.kernel,
    out_type=jax.ShapeDtypeStruct((_B, 6 * _D * 16), jnp.float32),
    mesh=plsc.VectorSubcoreMesh(core_axis_name="c", subcore_axis_name="s"),
    scratch_types=[
        pltpu.VMEM((2, 128, 128), jnp.float32),
        pltpu.VMEM((2, _DT, 16, 128), jnp.float32),
        pltpu.VMEM((6 * _D * 16,), jnp.float32),
        pltpu.SemaphoreType.DMA,
        pltpu.SemaphoreType.DMA,
    ],
)(_sc_pool_body)


def _finale_sc_body(t_ref, b_ref, cpc_ref, cpr_ref, tidc_ref, tidr_ref,
                    p_ref, out_ref):
    T = jnp.sum(t_ref[...], axis=2)
    Bg = jnp.sum(b_ref[...], axis=2)
    cntc = jnp.sum(cpc_ref[...], axis=1, keepdims=True)
    cntr = jnp.sum(cpr_ref[...], axis=0, keepdims=True)
    _finale_math(T, Bg, cntc, cntr, tidc_ref[...], tidr_ref[...], p_ref[...],
                 out_ref)


def _finale_body(t_ref, b_ref, cc_ref, cr_ref, tidc_ref, tidr_ref, p_ref,
                 out_ref):
    _finale_math(t_ref[...], b_ref[...], cc_ref[...], cr_ref[...],
                 tidc_ref[...], tidr_ref[...], p_ref[...], out_ref)


def _finale_math(T, Bg, cntc, cntr, tidc, tidr, P, out_ref):
    rt = T / jnp.maximum(cntc, 1.0)
    rt = rt / jnp.maximum(
        jnp.sqrt(jnp.sum(rt * rt, axis=1, keepdims=True)), 1e-12)
    rb = Bg / jnp.maximum(float(_HW) - cntc, 1.0)
    rb = rb / jnp.maximum(
        jnp.sqrt(jnp.sum(rb * rb, axis=1, keepdims=True)), 1e-12)

    gt = lax.dot_general(rt, rt, (((1,), (1,)), ((), ())),
                         preferred_element_type=jnp.float32)
    gb = lax.dot_general(rt, rb, (((1,), (1,)), ((), ())),
                         preferred_element_type=jnp.float32)

    rowi = lax.broadcasted_iota(jnp.int32, (_N2, _N2), 0)
    colj = lax.broadcasted_iota(jnp.int32, (_N2, _N2), 1)
    tri = (rowi <= colj).astype(jnp.float32)

    cooc = (tidc != tidr) & (cntr != 0.0)
    cf = cooc.astype(jnp.float32)
    csum = lax.dot_general(cf, tri, (((1,), (0,)), ((), ())),
                           preferred_element_type=jnp.float32)
    ndiff = csum[:, _N2 - 1:_N2]
    jf = colj.astype(jnp.float32)
    key = jnp.where(cooc, csum - 1.0, ndiff + jf - csum)

    pcond = (tidc == tidr) & (colj != tidc)
    pf = pcond.astype(jnp.float32)
    psum = lax.dot_general(pf, tri, (((1,), (0,)), ((), ())),
                           preferred_element_type=jnp.float32)
    onehot = pf * (psum == 1.0).astype(jnp.float32)
    has_pos = psum[:, _N2 - 1:_N2] > 0.0
    fallback = (colj == 0).astype(jnp.float32)
    oh = jnp.where(has_pos, onehot, fallback)
    pos_logit = jnp.sum(oh * gt, axis=1, keepdims=True)

    cols = []
    for k in range(N_NEGATIVES):
        pk = P[:, k:k + 1].astype(jnp.float32)
        sel = pk < ndiff
        g = jnp.where(sel, gt, gb)
        match = (key == pk)
        cols.append(jnp.sum(jnp.where(match, g, 0.0), axis=1, keepdims=True))
    nl = jnp.concatenate(cols, axis=1) / TEMPERATURE
    pos = pos_logit / TEMPERATURE
    m = jnp.max(nl, axis=1, keepdims=True)
    row_loss = jnp.log(jnp.sum(jnp.exp(nl - m), axis=1, keepdims=True)) \
        - (pos - m)
    out_ref[...] = jnp.sum(row_loss, axis=0, keepdims=True) / float(_N2)


@functools.partial(jax.jit, static_argnames=())
def _run(features, labels, task_ids, perms):
    pool = pl.pallas_call(
        _pool_body,
        grid=(_B,),
        in_specs=[
            pl.BlockSpec((1, _D, 128, 128), lambda i: (i, 0, 0, 0)),
            pl.BlockSpec((1, 2, 128, 128), lambda i: (i, 0, 0, 0)),
        ],
        out_specs=[
            pl.BlockSpec((1, 2, _D), lambda i: (i, 0, 0)),
            pl.BlockSpec((1, 2, _D), lambda i: (i, 0, 0)),
            pl.BlockSpec((1, 2, _D), lambda i: (i, 0, 0)),
        ],
        out_shape=[
            jax.ShapeDtypeStruct((_B, 2, _D), jnp.float32),
            jax.ShapeDtypeStruct((_B, 2, _D), jnp.float32),
            jax.ShapeDtypeStruct((_B, 2, _D), jnp.float32),
        ],
    )
    t_sums, b_sums, cnts = pool(features, labels)

    t2 = t_sums.reshape(_N2, _D)
    b2 = b_sums.reshape(_N2, _D)
    cnt = cnts[:, :, 0].reshape(_N2)

    finale = pl.pallas_call(
        _finale_body,
        out_shape=jax.ShapeDtypeStruct((1, 1), jnp.float32),
    )
    loss = finale(
        t2, b2,
        cnt.reshape(_N2, 1), cnt.reshape(1, _N2),
        task_ids.reshape(_N2, 1), task_ids.reshape(1, _N2),
        perms,
    )
    return loss[0, 0]


@jax.jit
def _run_sc(features, labels, task_ids, perms):
    o = _sc_pool(features, labels).reshape(_B, 6, _D, 16)
    t2 = o[:, 0:2].reshape(_N2, _D, 16)
    b2 = o[:, 2:4].reshape(_N2, _D, 16)
    cntp = o[:, 4:6, 0, :].reshape(_N2, 16)
    finale = pl.pallas_call(
        _finale_sc_body,
        out_shape=jax.ShapeDtypeStruct((1, 1), jnp.float32),
    )
    loss = finale(
        t2, b2, cntp, cntp.T,
        task_ids.reshape(_N2, 1), task_ids.reshape(1, _N2),
        perms,
    )
    return loss[0, 0]


_rng = np.random.default_rng(0)
_PERMS = np.stack(
    [_rng.permutation(_D)[:N_NEGATIVES] for _ in range(_N2)]).astype(np.int32)


def kernel(features, labels, tasks):
    task_ids = jnp.stack([2 * tasks, 2 * tasks + 1], axis=1).reshape(-1)
    return _run_sc(features, labels, task_ids.astype(jnp.int32), _PERMS)


# speedup vs baseline: 2.8090x; 2.8090x over previous
import functools

import jax
import jax.numpy as jnp
import numpy as np
from jax import lax
from jax.experimental import pallas as pl
from jax.experimental.pallas import tpu as pltpu
from jax.experimental.pallas import tpu_sc as plsc

TEMPERATURE = 0.07
N_NEGATIVES = 32
_B = 64
_D = 64
_HW = 128 * 128
_N2 = 2 * _B


def _pool_body(f_ref, l_ref, t_ref, b_ref, c_ref):
    f = f_ref[0]
    l0 = l_ref[0, 0]
    l1 = l_ref[0, 1]
    t0p = jnp.sum(f * l0[None, :, :], axis=1)
    t1p = jnp.sum(f * l1[None, :, :], axis=1)
    sp = jnp.sum(f, axis=1)
    t0 = jnp.sum(t0p, axis=1)
    t1 = jnp.sum(t1p, axis=1)
    s = jnp.sum(sp, axis=1)
    t_ref[0, 0] = t0
    t_ref[0, 1] = t1
    b_ref[0, 0] = s - t0
    b_ref[0, 1] = s - t1
    c_ref[0, 0] = jnp.broadcast_to(jnp.sum(l0), (_D,))
    c_ref[0, 1] = jnp.broadcast_to(jnp.sum(l1), (_D,))


_NC = 2
_NS = 16
_DT = 16
_NSC = 16
_NTC = _B - _NSC
_UPW = _NSC * (_D // _DT) // (_NC * _NS)


def _sc_split_body(f_hbm, l_hbm, out_hbm, lab_v, fbuf, out_v, sem0, sem1):
    wid = lax.axis_index("s") * _NC + lax.axis_index("c")

    def do_unit(k, carry):
        unit = wid * _UPW + k
        b = _NTC + (unit // 4)
        dt = unit % 4
        pltpu.sync_copy(l_hbm.at[b], lab_v)

        @pl.when(dt == 0)
        def _():
            def cnt_body(i, cc):
                c0, c1 = cc
                h = i >> 3
                w0 = (i & 7) * 16
                return (c0 + lab_v[0, h, pl.ds(w0, 16)],
                        c1 + lab_v[1, h, pl.ds(w0, 16)])

            c0v, c1v = lax.fori_loop(
                0, 1024, cnt_body,
                (jnp.zeros((16,), jnp.float32),
                 jnp.zeros((16,), jnp.float32)))
            out_v[pl.ds(4 * 16 * 16, 16)] = c0v
            out_v[pl.ds(4 * 16 * 16 + 16, 16)] = c1v
            pltpu.sync_copy(
                out_v.at[pl.ds(4 * 16 * 16, 16)],
                out_hbm.at[b, pl.ds(4 * _D * 16, 16)])
            pltpu.sync_copy(
                out_v.at[pl.ds(4 * 16 * 16 + 16, 16)],
                out_hbm.at[b, pl.ds(5 * _D * 16, 16)])

        sems = (sem0, sem1)
        copies = [None, None]
        copies[0] = pltpu.async_copy(
            f_hbm.at[b, pl.ds(dt * _DT, _DT), pl.ds(0, 16), :],
            fbuf.at[0], sem0)
        accs = tuple(jnp.zeros((16,), jnp.float32) for _ in range(3 * _DT))
        for hc in range(8):
            cur = hc % 2
            if hc < 7:
                nxt = (hc + 1) % 2
                copies[nxt] = pltpu.async_copy(
                    f_hbm.at[b, pl.ds(dt * _DT, _DT),
                             pl.ds((hc + 1) * 16, 16), :],
                    fbuf.at[nxt], sems[nxt])
            copies[cur].wait()

            def chunk_body(i, acc, _cur=cur, _hc=hc):
                h = i >> 3
                w0 = (i & 7) * 16
                l0c = lab_v[0, _hc * 16 + h, pl.ds(w0, 16)]
                l1c = lab_v[1, _hc * 16 + h, pl.ds(w0, 16)]
                t0s, t1s, ss = [], [], []
                for j in range(_DT):
                    fv = fbuf[_cur, j, h, pl.ds(w0, 16)]
                    t0s.append(acc[j] + fv * l0c)
                    t1s.append(acc[_DT + j] + fv * l1c)
                    ss.append(acc[2 * _DT + j] + fv)
                return tuple(t0s + t1s + ss)

            accs = lax.fori_loop(0, 128, chunk_body, accs)
        for j in range(_DT):
            out_v[pl.ds((0 * _DT + j) * 16, 16)] = accs[j]
            out_v[pl.ds((1 * _DT + j) * 16, 16)] = accs[_DT + j]
            out_v[pl.ds((2 * _DT + j) * 16, 16)] = \
                accs[2 * _DT + j] - accs[j]
            out_v[pl.ds((3 * _DT + j) * 16, 16)] = \
                accs[2 * _DT + j] - accs[_DT + j]
        for srow in range(4):
            pltpu.sync_copy(
                out_v.at[pl.ds(srow * 256, 256)],
                out_hbm.at[b, pl.ds((srow * _D + dt * _DT) * 16, 256)])
        return carry

    lax.fori_loop(0, _UPW, do_unit, 0)


_sc_split_pool = functools.partial(
    pl.kernel,
    out_type=jax.ShapeDtypeStruct((_B, 6 * _D * 16), jnp.float32),
    mesh=plsc.VectorSubcoreMesh(core_axis_name="c", subcore_axis_name="s"),
    scratch_types=[
        pltpu.VMEM((2, 128, 128), jnp.float32),
        pltpu.VMEM((2, _DT, 16, 128), jnp.float32),
        pltpu.VMEM((4 * 16 * 16 + 32,), jnp.float32),
        pltpu.SemaphoreType.DMA,
        pltpu.SemaphoreType.DMA,
    ],
)(_sc_split_body)


def _sc_pool_body(f_hbm, l_hbm, out_hbm, lab_v, fbuf, out_v, sem0, sem1):
    wid = lax.axis_index("s") * _NC + lax.axis_index("c")

    def do_image(img, carry):
        b = wid * 2 + img
        pltpu.sync_copy(l_hbm.at[b], lab_v)

        def cnt_body(i, cc):
            c0, c1 = cc
            h = i >> 3
            w0 = (i & 7) * 16
            return (c0 + lab_v[0, h, pl.ds(w0, 16)],
                    c1 + lab_v[1, h, pl.ds(w0, 16)])

        c0v, c1v = lax.fori_loop(
            0, 1024, cnt_body,
            (jnp.zeros((16,), jnp.float32), jnp.zeros((16,), jnp.float32)))
        out_v[pl.ds(4 * _D * 16, 16)] = c0v
        out_v[pl.ds(5 * _D * 16, 16)] = c1v

        sems = (sem0, sem1)
        for dt in range(_D // _DT):
            copies = [None, None]
            copies[0] = pltpu.async_copy(
                f_hbm.at[b, pl.ds(dt * _DT, _DT), pl.ds(0, 16), :],
                fbuf.at[0], sem0)
            accs = tuple(jnp.zeros((16,), jnp.float32) for _ in range(3 * _DT))
            for hc in range(8):
                cur = hc % 2
                if hc < 7:
                    nxt = (hc + 1) % 2
                    copies[nxt] = pltpu.async_copy(
                        f_hbm.at[b, pl.ds(dt * _DT, _DT),
                                 pl.ds((hc + 1) * 16, 16), :],
                        fbuf.at[nxt], sems[nxt])
                copies[cur].wait()

                def chunk_body(i, acc, _cur=cur, _hc=hc):
                    h = i >> 3
                    w0 = (i & 7) * 16
                    l0c = lab_v[0, _hc * 16 + h, pl.ds(w0, 16)]
                    l1c = lab_v[1, _hc * 16 + h, pl.ds(w0, 16)]
                    t0s, t1s, ss = [], [], []
                    for j in range(_DT):
                        fv = fbuf[_cur, j, h, pl.ds(w0, 16)]
                        t0s.append(acc[j] + fv * l0c)
                        t1s.append(acc[_DT + j] + fv * l1c)
                        ss.append(acc[2 * _DT + j] + fv)
                    return tuple(t0s + t1s + ss)

                accs = lax.fori_loop(0, 128, chunk_body, accs)
            for j in range(_DT):
                d = dt * _DT + j
                out_v[pl.ds((0 * _D + d) * 16, 16)] = accs[j]
                out_v[pl.ds((1 * _D + d) * 16, 16)] = accs[_DT + j]
                out_v[pl.ds((2 * _D + d) * 16, 16)] = \
                    accs[2 * _DT + j] - accs[j]
                out_v[pl.ds((3 * _D + d) * 16, 16)] = \
                    accs[2 * _DT + j] - accs[_DT + j]
        pltpu.sync_copy(out_v, out_hbm.at[b])
        return carry

    lax.fori_loop(0, 2, do_image, 0)


_sc_pool = functools.partial(
    pl.kernel,
    out_type=jax.ShapeDtypeStruct((_B, 6 * _D * 16), jnp.float32),
    mesh=plsc.VectorSubcoreMesh(core_axis_name="c", subcore_axis_name="s"),
    scratch_types=[
        pltpu.VMEM((2, 128, 128), jnp.float32),
        pltpu.VMEM((2, _DT, 16, 128), jnp.float32),
        pltpu.VMEM((6 * _D * 16,), jnp.float32),
        pltpu.SemaphoreType.DMA,
        pltpu.SemaphoreType.DMA,
    ],
)(_sc_pool_body)


def _finale_mix_body(ttc_ref, btc_ref, ctcc_ref, ctcr_ref, tsc_ref, bsc_ref,
                     cscp_ref, cscpt_ref, tidc_ref, tidr_ref, p_ref, out_ref):
    T = jnp.concatenate(
        [ttc_ref[...], jnp.sum(tsc_ref[...], axis=2)], axis=0)
    Bg = jnp.concatenate(
        [btc_ref[...], jnp.sum(bsc_ref[...], axis=2)], axis=0)
    cntc = jnp.concatenate(
        [ctcc_ref[...], jnp.sum(cscp_ref[...], axis=1, keepdims=True)],
        axis=0)
    cntr = jnp.concatenate(
        [ctcr_ref[...], jnp.sum(cscpt_ref[...], axis=0, keepdims=True)],
        axis=1)
    _finale_math(T, Bg, cntc, cntr, tidc_ref[...], tidr_ref[...], p_ref[...],
                 out_ref)


def _finale_sc_body(t_ref, b_ref, cpc_ref, cpr_ref, tidc_ref, tidr_ref,
                    p_ref, out_ref):
    T = jnp.sum(t_ref[...], axis=2)
    Bg = jnp.sum(b_ref[...], axis=2)
    cntc = jnp.sum(cpc_ref[...], axis=1, keepdims=True)
    cntr = jnp.sum(cpr_ref[...], axis=0, keepdims=True)
    _finale_math(T, Bg, cntc, cntr, tidc_ref[...], tidr_ref[...], p_ref[...],
                 out_ref)


def _finale_body(t_ref, b_ref, cc_ref, cr_ref, tidc_ref, tidr_ref, p_ref,
                 out_ref):
    _finale_math(t_ref[...], b_ref[...], cc_ref[...], cr_ref[...],
                 tidc_ref[...], tidr_ref[...], p_ref[...], out_ref)


def _finale_math(T, Bg, cntc, cntr, tidc, tidr, P, out_ref):
    rt = T / jnp.maximum(cntc, 1.0)
    rt = rt / jnp.maximum(
        jnp.sqrt(jnp.sum(rt * rt, axis=1, keepdims=True)), 1e-12)
    rb = Bg / jnp.maximum(float(_HW) - cntc, 1.0)
    rb = rb / jnp.maximum(
        jnp.sqrt(jnp.sum(rb * rb, axis=1, keepdims=True)), 1e-12)

    gt = lax.dot_general(rt, rt, (((1,), (1,)), ((), ())),
                         preferred_element_type=jnp.float32)
    gb = lax.dot_general(rt, rb, (((1,), (1,)), ((), ())),
                         preferred_element_type=jnp.float32)

    rowi = lax.broadcasted_iota(jnp.int32, (_N2, _N2), 0)
    colj = lax.broadcasted_iota(jnp.int32, (_N2, _N2), 1)
    tri = (rowi <= colj).astype(jnp.float32)

    cooc = (tidc != tidr) & (cntr != 0.0)
    cf = cooc.astype(jnp.float32)
    csum = lax.dot_general(cf, tri, (((1,), (0,)), ((), ())),
                           preferred_element_type=jnp.float32)
    ndiff = csum[:, _N2 - 1:_N2]
    jf = colj.astype(jnp.float32)
    key = jnp.where(cooc, csum - 1.0, ndiff + jf - csum)

    pcond = (tidc == tidr) & (colj != tidc)
    pf = pcond.astype(jnp.float32)
    psum = lax.dot_general(pf, tri, (((1,), (0,)), ((), ())),
                           preferred_element_type=jnp.float32)
    onehot = pf * (psum == 1.0).astype(jnp.float32)
    has_pos = psum[:, _N2 - 1:_N2] > 0.0
    fallback = (colj == 0).astype(jnp.float32)
    oh = jnp.where(has_pos, onehot, fallback)
    pos_logit = jnp.sum(oh * gt, axis=1, keepdims=True)

    cols = []
    for k in range(N_NEGATIVES):
        pk = P[:, k:k + 1].astype(jnp.float32)
        sel = pk < ndiff
        g = jnp.where(sel, gt, gb)
        match = (key == pk)
        cols.append(jnp.sum(jnp.where(match, g, 0.0), axis=1, keepdims=True))
    nl = jnp.concatenate(cols, axis=1) / TEMPERATURE
    pos = pos_logit / TEMPERATURE
    m = jnp.max(nl, axis=1, keepdims=True)
    row_loss = jnp.log(jnp.sum(jnp.exp(nl - m), axis=1, keepdims=True)) \
        - (pos - m)
    out_ref[...] = jnp.sum(row_loss, axis=0, keepdims=True) / float(_N2)


@functools.partial(jax.jit, static_argnames=())
def _run(features, labels, task_ids, perms):
    pool = pl.pallas_call(
        _pool_body,
        grid=(_B,),
        in_specs=[
            pl.BlockSpec((1, _D, 128, 128), lambda i: (i, 0, 0, 0)),
            pl.BlockSpec((1, 2, 128, 128), lambda i: (i, 0, 0, 0)),
        ],
        out_specs=[
            pl.BlockSpec((1, 2, _D), lambda i: (i, 0, 0)),
            pl.BlockSpec((1, 2, _D), lambda i: (i, 0, 0)),
            pl.BlockSpec((1, 2, _D), lambda i: (i, 0, 0)),
        ],
        out_shape=[
            jax.ShapeDtypeStruct((_B, 2, _D), jnp.float32),
            jax.ShapeDtypeStruct((_B, 2, _D), jnp.float32),
            jax.ShapeDtypeStruct((_B, 2, _D), jnp.float32),
        ],
    )
    t_sums, b_sums, cnts = pool(features, labels)

    t2 = t_sums.reshape(_N2, _D)
    b2 = b_sums.reshape(_N2, _D)
    cnt = cnts[:, :, 0].reshape(_N2)

    finale = pl.pallas_call(
        _finale_body,
        out_shape=jax.ShapeDtypeStruct((1, 1), jnp.float32),
    )
    loss = finale(
        t2, b2,
        cnt.reshape(_N2, 1), cnt.reshape(1, _N2),
        task_ids.reshape(_N2, 1), task_ids.reshape(1, _N2),
        perms,
    )
    return loss[0, 0]


@jax.jit
def _run_mix(features, labels, task_ids, perms):
    pool = pl.pallas_call(
        _pool_body,
        grid=(_NTC,),
        in_specs=[
            pl.BlockSpec((1, _D, 128, 128), lambda i: (i, 0, 0, 0)),
            pl.BlockSpec((1, 2, 128, 128), lambda i: (i, 0, 0, 0)),
        ],
        out_specs=[
            pl.BlockSpec((1, 2, _D), lambda i: (i, 0, 0)),
            pl.BlockSpec((1, 2, _D), lambda i: (i, 0, 0)),
            pl.BlockSpec((1, 2, _D), lambda i: (i, 0, 0)),
        ],
        out_shape=[
            jax.ShapeDtypeStruct((_NTC, 2, _D), jnp.float32),
            jax.ShapeDtypeStruct((_NTC, 2, _D), jnp.float32),
            jax.ShapeDtypeStruct((_NTC, 2, _D), jnp.float32),
        ],
    )
    t_sums, b_sums, cnts = pool(features, labels)
    osc = _sc_split_pool(features, labels).reshape(_B, 6, _D, 16)

    ttc = t_sums.reshape(2 * _NTC, _D)
    btc = b_sums.reshape(2 * _NTC, _D)
    ctc = cnts[:, :, 0].reshape(2 * _NTC)
    tsc = osc[_NTC:, 0:2].reshape(2 * _NSC, _D, 16)
    bsc = osc[_NTC:, 2:4].reshape(2 * _NSC, _D, 16)
    csc = osc[_NTC:, 4:6, 0, :].reshape(2 * _NSC, 16)

    finale = pl.pallas_call(
        _finale_mix_body,
        out_shape=jax.ShapeDtypeStruct((1, 1), jnp.float32),
    )
    loss = finale(
        ttc, btc, ctc.reshape(2 * _NTC, 1), ctc.reshape(1, 2 * _NTC),
        tsc, bsc, csc, csc.T,
        task_ids.reshape(_N2, 1), task_ids.reshape(1, _N2),
        perms,
    )
    return loss[0, 0]


@jax.jit
def _run_sc(features, labels, task_ids, perms):
    o = _sc_pool(features, labels).reshape(_B, 6, _D, 16)
    t2 = o[:, 0:2].reshape(_N2, _D, 16)
    b2 = o[:, 2:4].reshape(_N2, _D, 16)
    cntp = o[:, 4:6, 0, :].reshape(_N2, 16)
    finale = pl.pallas_call(
        _finale_sc_body,
        out_shape=jax.ShapeDtypeStruct((1, 1), jnp.float32),
    )
    loss = finale(
        t2, b2, cntp, cntp.T,
        task_ids.reshape(_N2, 1), task_ids.reshape(1, _N2),
        perms,
    )
    return loss[0, 0]


_rng = np.random.default_rng(0)
_PERMS = np.stack(
    [_rng.permutation(_D)[:N_NEGATIVES] for _ in range(_N2)]).astype(np.int32)


def kernel(features, labels, tasks):
    task_ids = jnp.stack([2 * tasks, 2 * tasks + 1], axis=1).reshape(-1)
    return _run_mix(features, labels, task_ids.astype(jnp.int32), _PERMS)


# speedup vs baseline: 2.8110x; 1.0007x over previous
import functools

import jax
import jax.numpy as jnp
import numpy as np
from jax import lax
from jax.experimental import pallas as pl
from jax.experimental.pallas import tpu as pltpu
from jax.experimental.pallas import tpu_sc as plsc

TEMPERATURE = 0.07
N_NEGATIVES = 32
_B = 64
_D = 64
_HW = 128 * 128
_N2 = 2 * _B


def _pool_body(f_ref, l_ref, t_ref, b_ref, c_ref):
    f = f_ref[0]
    l0 = l_ref[0, 0]
    l1 = l_ref[0, 1]
    t0p = jnp.sum(f * l0[None, :, :], axis=1)
    t1p = jnp.sum(f * l1[None, :, :], axis=1)
    sp = jnp.sum(f, axis=1)
    t0 = jnp.sum(t0p, axis=1)
    t1 = jnp.sum(t1p, axis=1)
    s = jnp.sum(sp, axis=1)
    t_ref[0, 0] = t0
    t_ref[0, 1] = t1
    b_ref[0, 0] = s - t0
    b_ref[0, 1] = s - t1
    c_ref[0, 0] = jnp.broadcast_to(jnp.sum(l0), (_D,))
    c_ref[0, 1] = jnp.broadcast_to(jnp.sum(l1), (_D,))


_NC = 2
_NS = 16
_DT = 16
_NSC = 16
_NTC = _B - _NSC
_UPW = _NSC * (_D // _DT) // (_NC * _NS)


def _sc_split_body(f_hbm, l_hbm, out_hbm, lab_v, fbuf, out_v, sem0, sem1):
    wid = lax.axis_index("s") * _NC + lax.axis_index("c")

    def do_unit(k, carry):
        unit = wid * _UPW + k
        b = _NTC + (unit // 4)
        dt = unit % 4
        pltpu.sync_copy(l_hbm.at[b], lab_v)

        @pl.when(dt == 0)
        def _():
            def cnt_body(i, cc):
                c0, c1 = cc
                h = i >> 3
                w0 = (i & 7) * 16
                return (c0 + lab_v[0, h, pl.ds(w0, 16)],
                        c1 + lab_v[1, h, pl.ds(w0, 16)])

            c0v, c1v = lax.fori_loop(
                0, 1024, cnt_body,
                (jnp.zeros((16,), jnp.float32),
                 jnp.zeros((16,), jnp.float32)))
            out_v[pl.ds(4 * 16 * 16, 16)] = c0v
            out_v[pl.ds(4 * 16 * 16 + 16, 16)] = c1v
            pltpu.sync_copy(
                out_v.at[pl.ds(4 * 16 * 16, 16)],
                out_hbm.at[b, pl.ds(4 * _D * 16, 16)])
            pltpu.sync_copy(
                out_v.at[pl.ds(4 * 16 * 16 + 16, 16)],
                out_hbm.at[b, pl.ds(5 * _D * 16, 16)])

        sems = (sem0, sem1)
        copies = [None, None]
        copies[0] = pltpu.async_copy(
            f_hbm.at[b, pl.ds(dt * _DT, _DT), pl.ds(0, 16), :],
            fbuf.at[0], sem0)
        accs = tuple(jnp.zeros((16,), jnp.float32) for _ in range(3 * _DT))
        for hc in range(8):
            cur = hc % 2
            if hc < 7:
                nxt = (hc + 1) % 2
                copies[nxt] = pltpu.async_copy(
                    f_hbm.at[b, pl.ds(dt * _DT, _DT),
                             pl.ds((hc + 1) * 16, 16), :],
                    fbuf.at[nxt], sems[nxt])
            copies[cur].wait()

            def chunk_body(i, acc, _cur=cur, _hc=hc):
                h = i >> 3
                w0 = (i & 7) * 16
                l0c = lab_v[0, _hc * 16 + h, pl.ds(w0, 16)]
                l1c = lab_v[1, _hc * 16 + h, pl.ds(w0, 16)]
                t0s, t1s, ss = [], [], []
                for j in range(_DT):
                    fv = fbuf[_cur, j, h, pl.ds(w0, 16)]
                    t0s.append(acc[j] + fv * l0c)
                    t1s.append(acc[_DT + j] + fv * l1c)
                    ss.append(acc[2 * _DT + j] + fv)
                return tuple(t0s + t1s + ss)

            accs = lax.fori_loop(0, 128, chunk_body, accs)
        for j in range(_DT):
            out_v[pl.ds((0 * _DT + j) * 16, 16)] = accs[j]
            out_v[pl.ds((1 * _DT + j) * 16, 16)] = accs[_DT + j]
            out_v[pl.ds((2 * _DT + j) * 16, 16)] = \
                accs[2 * _DT + j] - accs[j]
            out_v[pl.ds((3 * _DT + j) * 16, 16)] = \
                accs[2 * _DT + j] - accs[_DT + j]
        for srow in range(4):
            pltpu.sync_copy(
                out_v.at[pl.ds(srow * 256, 256)],
                out_hbm.at[b, pl.ds((srow * _D + dt * _DT) * 16, 256)])
        return carry

    lax.fori_loop(0, _UPW, do_unit, 0)


_sc_split_pool = functools.partial(
    pl.kernel,
    out_type=jax.ShapeDtypeStruct((_B, 6 * _D * 16), jnp.float32),
    mesh=plsc.VectorSubcoreMesh(core_axis_name="c", subcore_axis_name="s"),
    scratch_types=[
        pltpu.VMEM((2, 128, 128), jnp.float32),
        pltpu.VMEM((2, _DT, 16, 128), jnp.float32),
        pltpu.VMEM((4 * 16 * 16 + 32,), jnp.float32),
        pltpu.SemaphoreType.DMA,
        pltpu.SemaphoreType.DMA,
    ],
)(_sc_split_body)


def _sc_pool_body(f_hbm, l_hbm, out_hbm, lab_v, fbuf, out_v, sem0, sem1):
    wid = lax.axis_index("s") * _NC + lax.axis_index("c")

    def do_image(img, carry):
        b = wid * 2 + img
        pltpu.sync_copy(l_hbm.at[b], lab_v)

        def cnt_body(i, cc):
            c0, c1 = cc
            h = i >> 3
            w0 = (i & 7) * 16
            return (c0 + lab_v[0, h, pl.ds(w0, 16)],
                    c1 + lab_v[1, h, pl.ds(w0, 16)])

        c0v, c1v = lax.fori_loop(
            0, 1024, cnt_body,
            (jnp.zeros((16,), jnp.float32), jnp.zeros((16,), jnp.float32)))
        out_v[pl.ds(4 * _D * 16, 16)] = c0v
        out_v[pl.ds(5 * _D * 16, 16)] = c1v

        sems = (sem0, sem1)
        for dt in range(_D // _DT):
            copies = [None, None]
            copies[0] = pltpu.async_copy(
                f_hbm.at[b, pl.ds(dt * _DT, _DT), pl.ds(0, 16), :],
                fbuf.at[0], sem0)
            accs = tuple(jnp.zeros((16,), jnp.float32) for _ in range(3 * _DT))
            for hc in range(8):
                cur = hc % 2
                if hc < 7:
                    nxt = (hc + 1) % 2
                    copies[nxt] = pltpu.async_copy(
                        f_hbm.at[b, pl.ds(dt * _DT, _DT),
                                 pl.ds((hc + 1) * 16, 16), :],
                        fbuf.at[nxt], sems[nxt])
                copies[cur].wait()

                def chunk_body(i, acc, _cur=cur, _hc=hc):
                    h = i >> 3
                    w0 = (i & 7) * 16
                    l0c = lab_v[0, _hc * 16 + h, pl.ds(w0, 16)]
                    l1c = lab_v[1, _hc * 16 + h, pl.ds(w0, 16)]
                    t0s, t1s, ss = [], [], []
                    for j in range(_DT):
                        fv = fbuf[_cur, j, h, pl.ds(w0, 16)]
                        t0s.append(acc[j] + fv * l0c)
                        t1s.append(acc[_DT + j] + fv * l1c)
                        ss.append(acc[2 * _DT + j] + fv)
                    return tuple(t0s + t1s + ss)

                accs = lax.fori_loop(0, 128, chunk_body, accs)
            for j in range(_DT):
                d = dt * _DT + j
                out_v[pl.ds((0 * _D + d) * 16, 16)] = accs[j]
                out_v[pl.ds((1 * _D + d) * 16, 16)] = accs[_DT + j]
                out_v[pl.ds((2 * _D + d) * 16, 16)] = \
                    accs[2 * _DT + j] - accs[j]
                out_v[pl.ds((3 * _D + d) * 16, 16)] = \
                    accs[2 * _DT + j] - accs[_DT + j]
        pltpu.sync_copy(out_v, out_hbm.at[b])
        return carry

    lax.fori_loop(0, 2, do_image, 0)


_sc_pool = functools.partial(
    pl.kernel,
    out_type=jax.ShapeDtypeStruct((_B, 6 * _D * 16), jnp.float32),
    mesh=plsc.VectorSubcoreMesh(core_axis_name="c", subcore_axis_name="s"),
    scratch_types=[
        pltpu.VMEM((2, 128, 128), jnp.float32),
        pltpu.VMEM((2, _DT, 16, 128), jnp.float32),
        pltpu.VMEM((6 * _D * 16,), jnp.float32),
        pltpu.SemaphoreType.DMA,
        pltpu.SemaphoreType.DMA,
    ],
)(_sc_pool_body)


def _finale_mix_body(ttc_ref, btc_ref, ctcc_ref, ctcr_ref, tsc_ref, bsc_ref,
                     cscp_ref, cscpt_ref, tidc_ref, tidr_ref, p_ref, out_ref):
    T = jnp.concatenate(
        [ttc_ref[...], jnp.sum(tsc_ref[...], axis=2)], axis=0)
    Bg = jnp.concatenate(
        [btc_ref[...], jnp.sum(bsc_ref[...], axis=2)], axis=0)
    cntc = jnp.concatenate(
        [ctcc_ref[...], jnp.sum(cscp_ref[...], axis=1, keepdims=True)],
        axis=0)
    cntr = jnp.concatenate(
        [ctcr_ref[...], jnp.sum(cscpt_ref[...], axis=0, keepdims=True)],
        axis=1)
    _finale_math(T, Bg, cntc, cntr, tidc_ref[...], tidr_ref[...], p_ref[...],
                 out_ref)


def _finale_sc_body(t_ref, b_ref, cpc_ref, cpr_ref, tidc_ref, tidr_ref,
                    p_ref, out_ref):
    T = jnp.sum(t_ref[...], axis=2)
    Bg = jnp.sum(b_ref[...], axis=2)
    cntc = jnp.sum(cpc_ref[...], axis=1, keepdims=True)
    cntr = jnp.sum(cpr_ref[...], axis=0, keepdims=True)
    _finale_math(T, Bg, cntc, cntr, tidc_ref[...], tidr_ref[...], p_ref[...],
                 out_ref)


def _finale_body(t_ref, b_ref, cc_ref, cr_ref, tidc_ref, tidr_ref, p_ref,
                 out_ref):
    _finale_math(t_ref[...], b_ref[...], cc_ref[...], cr_ref[...],
                 tidc_ref[...], tidr_ref[...], p_ref[...], out_ref)


def _finale_math(T, Bg, cntc, cntr, tidc, tidr, P, out_ref):
    rt = T / jnp.maximum(cntc, 1.0)
    rt = rt / jnp.maximum(
        jnp.sqrt(jnp.sum(rt * rt, axis=1, keepdims=True)), 1e-12)
    rb = Bg / jnp.maximum(float(_HW) - cntc, 1.0)
    rb = rb / jnp.maximum(
        jnp.sqrt(jnp.sum(rb * rb, axis=1, keepdims=True)), 1e-12)

    gt = lax.dot_general(rt, rt, (((1,), (1,)), ((), ())),
                         preferred_element_type=jnp.float32)
    gb = lax.dot_general(rt, rb, (((1,), (1,)), ((), ())),
                         preferred_element_type=jnp.float32)

    rowi = lax.broadcasted_iota(jnp.int32, (_N2, _N2), 0)
    colj = lax.broadcasted_iota(jnp.int32, (_N2, _N2), 1)
    tri = (rowi <= colj).astype(jnp.float32)

    cooc = (tidc != tidr) & (cntr != 0.0)
    cf = cooc.astype(jnp.float32)
    csum = lax.dot_general(cf, tri, (((1,), (0,)), ((), ())),
                           preferred_element_type=jnp.float32)
    ndiff = csum[:, _N2 - 1:_N2]
    jf = colj.astype(jnp.float32)
    key = jnp.where(cooc, csum - 1.0, ndiff + jf - csum)

    pcond = (tidc == tidr) & (colj != tidc)
    pf = pcond.astype(jnp.float32)
    psum = lax.dot_general(pf, tri, (((1,), (0,)), ((), ())),
                           preferred_element_type=jnp.float32)
    onehot = pf * (psum == 1.0).astype(jnp.float32)
    has_pos = psum[:, _N2 - 1:_N2] > 0.0
    fallback = (colj == 0).astype(jnp.float32)
    oh = jnp.where(has_pos, onehot, fallback)
    pos_logit = jnp.sum(oh * gt, axis=1, keepdims=True)

    cols = []
    for k in range(N_NEGATIVES):
        pk = P[:, k:k + 1].astype(jnp.float32)
        sel = pk < ndiff
        g = jnp.where(sel, gt, gb)
        match = (key == pk)
        cols.append(jnp.sum(jnp.where(match, g, 0.0), axis=1, keepdims=True))
    nl = jnp.concatenate(cols, axis=1) / TEMPERATURE
    pos = pos_logit / TEMPERATURE
    m = jnp.max(nl, axis=1, keepdims=True)
    row_loss = jnp.log(jnp.sum(jnp.exp(nl - m), axis=1, keepdims=True)) \
        - (pos - m)
    out_ref[...] = jnp.sum(row_loss, axis=0, keepdims=True) / float(_N2)


@functools.partial(jax.jit, static_argnames=())
def _run(features, labels, task_ids, perms):
    pool = pl.pallas_call(
        _pool_body,
        grid=(_B,),
        in_specs=[
            pl.BlockSpec((1, _D, 128, 128), lambda i: (i, 0, 0, 0)),
            pl.BlockSpec((1, 2, 128, 128), lambda i: (i, 0, 0, 0)),
        ],
        out_specs=[
            pl.BlockSpec((1, 2, _D), lambda i: (i, 0, 0)),
            pl.BlockSpec((1, 2, _D), lambda i: (i, 0, 0)),
            pl.BlockSpec((1, 2, _D), lambda i: (i, 0, 0)),
        ],
        out_shape=[
            jax.ShapeDtypeStruct((_B, 2, _D), jnp.float32),
            jax.ShapeDtypeStruct((_B, 2, _D), jnp.float32),
            jax.ShapeDtypeStruct((_B, 2, _D), jnp.float32),
        ],
    )
    t_sums, b_sums, cnts = pool(features, labels)

    t2 = t_sums.reshape(_N2, _D)
    b2 = b_sums.reshape(_N2, _D)
    cnt = cnts[:, :, 0].reshape(_N2)

    finale = pl.pallas_call(
        _finale_body,
        out_shape=jax.ShapeDtypeStruct((1, 1), jnp.float32),
    )
    loss = finale(
        t2, b2,
        cnt.reshape(_N2, 1), cnt.reshape(1, _N2),
        task_ids.reshape(_N2, 1), task_ids.reshape(1, _N2),
        perms,
    )
    return loss[0, 0]


@jax.jit
def _run_mix(features, labels, task_ids, perms):
    osc = _sc_split_pool(features, labels).reshape(_B, 6, _D, 16)
    pool = pl.pallas_call(
        _pool_body,
        grid=(_NTC,),
        in_specs=[
            pl.BlockSpec((1, _D, 128, 128), lambda i: (i, 0, 0, 0)),
            pl.BlockSpec((1, 2, 128, 128), lambda i: (i, 0, 0, 0)),
        ],
        out_specs=[
            pl.BlockSpec((1, 2, _D), lambda i: (i, 0, 0)),
            pl.BlockSpec((1, 2, _D), lambda i: (i, 0, 0)),
            pl.BlockSpec((1, 2, _D), lambda i: (i, 0, 0)),
        ],
        out_shape=[
            jax.ShapeDtypeStruct((_NTC, 2, _D), jnp.float32),
            jax.ShapeDtypeStruct((_NTC, 2, _D), jnp.float32),
            jax.ShapeDtypeStruct((_NTC, 2, _D), jnp.float32),
        ],
    )
    t_sums, b_sums, cnts = pool(features, labels)

    ttc = t_sums.reshape(2 * _NTC, _D)
    btc = b_sums.reshape(2 * _NTC, _D)
    ctc = cnts[:, :, 0].reshape(2 * _NTC)
    tsc = osc[_NTC:, 0:2].reshape(2 * _NSC, _D, 16)
    bsc = osc[_NTC:, 2:4].reshape(2 * _NSC, _D, 16)
    csc = osc[_NTC:, 4:6, 0, :].reshape(2 * _NSC, 16)

    finale = pl.pallas_call(
        _finale_mix_body,
        out_shape=jax.ShapeDtypeStruct((1, 1), jnp.float32),
    )
    loss = finale(
        ttc, btc, ctc.reshape(2 * _NTC, 1), ctc.reshape(1, 2 * _NTC),
        tsc, bsc, csc, csc.T,
        task_ids.reshape(_N2, 1), task_ids.reshape(1, _N2),
        perms,
    )
    return loss[0, 0]


@jax.jit
def _run_sc(features, labels, task_ids, perms):
    o = _sc_pool(features, labels).reshape(_B, 6, _D, 16)
    t2 = o[:, 0:2].reshape(_N2, _D, 16)
    b2 = o[:, 2:4].reshape(_N2, _D, 16)
    cntp = o[:, 4:6, 0, :].reshape(_N2, 16)
    finale = pl.pallas_call(
        _finale_sc_body,
        out_shape=jax.ShapeDtypeStruct((1, 1), jnp.float32),
    )
    loss = finale(
        t2, b2, cntp, cntp.T,
        task_ids.reshape(_N2, 1), task_ids.reshape(1, _N2),
        perms,
    )
    return loss[0, 0]


_rng = np.random.default_rng(0)
_PERMS = np.stack(
    [_rng.permutation(_D)[:N_NEGATIVES] for _ in range(_N2)]).astype(np.int32)


def kernel(features, labels, tasks):
    task_ids = jnp.stack([2 * tasks, 2 * tasks + 1], axis=1).reshape(-1)
    return _run_mix(features, labels, task_ids.astype(jnp.int32), _PERMS)


# speedup vs baseline: 2.8833x; 1.0257x over previous
import functools

import jax
import jax.numpy as jnp
import numpy as np
from jax import lax
from jax.experimental import pallas as pl
from jax.experimental.pallas import tpu as pltpu
from jax.experimental.pallas import tpu_sc as plsc

TEMPERATURE = 0.07
N_NEGATIVES = 32
_B = 64
_D = 64
_HW = 128 * 128
_N2 = 2 * _B


def _pool_body(f_ref, l_ref, t_ref, b_ref, c_ref):
    f = f_ref[0]
    l0 = l_ref[0, 0]
    l1 = l_ref[0, 1]
    t0p = jnp.sum(f * l0[None, :, :], axis=1)
    t1p = jnp.sum(f * l1[None, :, :], axis=1)
    sp = jnp.sum(f, axis=1)
    t0 = jnp.sum(t0p, axis=1)
    t1 = jnp.sum(t1p, axis=1)
    s = jnp.sum(sp, axis=1)
    t_ref[0, 0] = t0
    t_ref[0, 1] = t1
    b_ref[0, 0] = s - t0
    b_ref[0, 1] = s - t1
    c_ref[0, 0] = jnp.broadcast_to(jnp.sum(l0), (_D,))
    c_ref[0, 1] = jnp.broadcast_to(jnp.sum(l1), (_D,))


_NC = 2
_NS = 16
_DT = 16
_NSC = 16
_NTC = _B - _NSC
_UPW = _NSC * (_D // _DT) // (_NC * _NS)


def _sc_split_body(f_hbm, l_hbm, out_hbm, lab_v, fbuf, out_v, sem0, sem1):
    wid = lax.axis_index("s") * _NC + lax.axis_index("c")

    def do_unit(k, carry):
        unit = wid * _UPW + k
        bo = unit // 4
        b = _NTC + bo
        dt = unit % 4
        pltpu.sync_copy(l_hbm.at[b], lab_v)

        @pl.when(dt == 0)
        def _():
            def cnt_body(i, cc):
                c0, c1 = cc
                h = i >> 3
                w0 = (i & 7) * 16
                return (c0 + lab_v[0, h, pl.ds(w0, 16)],
                        c1 + lab_v[1, h, pl.ds(w0, 16)])

            c0v, c1v = plsc.parallel_loop(
                0, 1024, unroll=4,
                carry=(jnp.zeros((16,), jnp.float32),
                       jnp.zeros((16,), jnp.float32)))(cnt_body)
            out_v[pl.ds(4 * 16 * 16, 16)] = c0v
            out_v[pl.ds(4 * 16 * 16 + 16, 16)] = c1v
            pltpu.sync_copy(
                out_v.at[pl.ds(4 * 16 * 16, 16)],
                out_hbm.at[bo, pl.ds(4 * _D * 16, 16)])
            pltpu.sync_copy(
                out_v.at[pl.ds(4 * 16 * 16 + 16, 16)],
                out_hbm.at[bo, pl.ds(5 * _D * 16, 16)])

        sems = (sem0, sem1)
        copies = [None, None]
        copies[0] = pltpu.async_copy(
            f_hbm.at[b, pl.ds(dt * _DT, _DT), pl.ds(0, 16), :],
            fbuf.at[0], sem0)
        accs = tuple(jnp.zeros((16,), jnp.float32) for _ in range(3 * _DT))
        for hc in range(8):
            cur = hc % 2
            if hc < 7:
                nxt = (hc + 1) % 2
                copies[nxt] = pltpu.async_copy(
                    f_hbm.at[b, pl.ds(dt * _DT, _DT),
                             pl.ds((hc + 1) * 16, 16), :],
                    fbuf.at[nxt], sems[nxt])
            copies[cur].wait()

            def chunk_body(i, acc, _cur=cur, _hc=hc):
                h = i >> 3
                w0 = (i & 7) * 16
                l0c = lab_v[0, _hc * 16 + h, pl.ds(w0, 16)]
                l1c = lab_v[1, _hc * 16 + h, pl.ds(w0, 16)]
                t0s, t1s, ss = [], [], []
                for j in range(_DT):
                    fv = fbuf[_cur, j, h, pl.ds(w0, 16)]
                    t0s.append(acc[j] + fv * l0c)
                    t1s.append(acc[_DT + j] + fv * l1c)
                    ss.append(acc[2 * _DT + j] + fv)
                return tuple(t0s + t1s + ss)

            accs = plsc.parallel_loop(
                0, 128, unroll=4, carry=accs)(chunk_body)
        for j in range(_DT):
            out_v[pl.ds((0 * _DT + j) * 16, 16)] = accs[j]
            out_v[pl.ds((1 * _DT + j) * 16, 16)] = accs[_DT + j]
            out_v[pl.ds((2 * _DT + j) * 16, 16)] = \
                accs[2 * _DT + j] - accs[j]
            out_v[pl.ds((3 * _DT + j) * 16, 16)] = \
                accs[2 * _DT + j] - accs[_DT + j]
        for srow in range(4):
            pltpu.sync_copy(
                out_v.at[pl.ds(srow * 256, 256)],
                out_hbm.at[bo, pl.ds((srow * _D + dt * _DT) * 16, 256)])
        return carry

    lax.fori_loop(0, _UPW, do_unit, 0)


_sc_split_pool = functools.partial(
    pl.kernel,
    out_type=jax.ShapeDtypeStruct((_NSC, 6 * _D * 16), jnp.float32),
    mesh=plsc.VectorSubcoreMesh(core_axis_name="c", subcore_axis_name="s"),
    scratch_types=[
        pltpu.VMEM((2, 128, 128), jnp.float32),
        pltpu.VMEM((2, _DT, 16, 128), jnp.float32),
        pltpu.VMEM((4 * 16 * 16 + 32,), jnp.float32),
        pltpu.SemaphoreType.DMA,
        pltpu.SemaphoreType.DMA,
    ],
)(_sc_split_body)


def _sc_pool_body(f_hbm, l_hbm, out_hbm, lab_v, fbuf, out_v, sem0, sem1):
    wid = lax.axis_index("s") * _NC + lax.axis_index("c")

    def do_image(img, carry):
        b = wid * 2 + img
        pltpu.sync_copy(l_hbm.at[b], lab_v)

        def cnt_body(i, cc):
            c0, c1 = cc
            h = i >> 3
            w0 = (i & 7) * 16
            return (c0 + lab_v[0, h, pl.ds(w0, 16)],
                    c1 + lab_v[1, h, pl.ds(w0, 16)])

        c0v, c1v = lax.fori_loop(
            0, 1024, cnt_body,
            (jnp.zeros((16,), jnp.float32), jnp.zeros((16,), jnp.float32)))
        out_v[pl.ds(4 * _D * 16, 16)] = c0v
        out_v[pl.ds(5 * _D * 16, 16)] = c1v

        sems = (sem0, sem1)
        for dt in range(_D // _DT):
            copies = [None, None]
            copies[0] = pltpu.async_copy(
                f_hbm.at[b, pl.ds(dt * _DT, _DT), pl.ds(0, 16), :],
                fbuf.at[0], sem0)
            accs = tuple(jnp.zeros((16,), jnp.float32) for _ in range(3 * _DT))
            for hc in range(8):
                cur = hc % 2
                if hc < 7:
                    nxt = (hc + 1) % 2
                    copies[nxt] = pltpu.async_copy(
                        f_hbm.at[b, pl.ds(dt * _DT, _DT),
                                 pl.ds((hc + 1) * 16, 16), :],
                        fbuf.at[nxt], sems[nxt])
                copies[cur].wait()

                def chunk_body(i, acc, _cur=cur, _hc=hc):
                    h = i >> 3
                    w0 = (i & 7) * 16
                    l0c = lab_v[0, _hc * 16 + h, pl.ds(w0, 16)]
                    l1c = lab_v[1, _hc * 16 + h, pl.ds(w0, 16)]
                    t0s, t1s, ss = [], [], []
                    for j in range(_DT):
                        fv = fbuf[_cur, j, h, pl.ds(w0, 16)]
                        t0s.append(acc[j] + fv * l0c)
                        t1s.append(acc[_DT + j] + fv * l1c)
                        ss.append(acc[2 * _DT + j] + fv)
                    return tuple(t0s + t1s + ss)

                accs = lax.fori_loop(0, 128, chunk_body, accs)
            for j in range(_DT):
                d = dt * _DT + j
                out_v[pl.ds((0 * _D + d) * 16, 16)] = accs[j]
                out_v[pl.ds((1 * _D + d) * 16, 16)] = accs[_DT + j]
                out_v[pl.ds((2 * _D + d) * 16, 16)] = \
                    accs[2 * _DT + j] - accs[j]
                out_v[pl.ds((3 * _D + d) * 16, 16)] = \
                    accs[2 * _DT + j] - accs[_DT + j]
        pltpu.sync_copy(out_v, out_hbm.at[b])
        return carry

    lax.fori_loop(0, 2, do_image, 0)


_sc_pool = functools.partial(
    pl.kernel,
    out_type=jax.ShapeDtypeStruct((_B, 6 * _D * 16), jnp.float32),
    mesh=plsc.VectorSubcoreMesh(core_axis_name="c", subcore_axis_name="s"),
    scratch_types=[
        pltpu.VMEM((2, 128, 128), jnp.float32),
        pltpu.VMEM((2, _DT, 16, 128), jnp.float32),
        pltpu.VMEM((6 * _D * 16,), jnp.float32),
        pltpu.SemaphoreType.DMA,
        pltpu.SemaphoreType.DMA,
    ],
)(_sc_pool_body)


def _finale_mix_body(ttc_ref, btc_ref, ctcc_ref, ctcr_ref, tsc_ref, bsc_ref,
                     cscp_ref, cscpt_ref, tidc_ref, tidr_ref, p_ref, out_ref):
    T = jnp.concatenate(
        [ttc_ref[...], jnp.sum(tsc_ref[...], axis=2)], axis=0)
    Bg = jnp.concatenate(
        [btc_ref[...], jnp.sum(bsc_ref[...], axis=2)], axis=0)
    cntc = jnp.concatenate(
        [ctcc_ref[...], jnp.sum(cscp_ref[...], axis=1, keepdims=True)],
        axis=0)
    cntr = jnp.concatenate(
        [ctcr_ref[...], jnp.sum(cscpt_ref[...], axis=0, keepdims=True)],
        axis=1)
    _finale_math(T, Bg, cntc, cntr, tidc_ref[...], tidr_ref[...], p_ref[...],
                 out_ref)


def _finale_sc_body(t_ref, b_ref, cpc_ref, cpr_ref, tidc_ref, tidr_ref,
                    p_ref, out_ref):
    T = jnp.sum(t_ref[...], axis=2)
    Bg = jnp.sum(b_ref[...], axis=2)
    cntc = jnp.sum(cpc_ref[...], axis=1, keepdims=True)
    cntr = jnp.sum(cpr_ref[...], axis=0, keepdims=True)
    _finale_math(T, Bg, cntc, cntr, tidc_ref[...], tidr_ref[...], p_ref[...],
                 out_ref)


def _finale_body(t_ref, b_ref, cc_ref, cr_ref, tidc_ref, tidr_ref, p_ref,
                 out_ref):
    _finale_math(t_ref[...], b_ref[...], cc_ref[...], cr_ref[...],
                 tidc_ref[...], tidr_ref[...], p_ref[...], out_ref)


def _finale_math(T, Bg, cntc, cntr, tidc, tidr, P, out_ref):
    rt = T / jnp.maximum(cntc, 1.0)
    rt = rt / jnp.maximum(
        jnp.sqrt(jnp.sum(rt * rt, axis=1, keepdims=True)), 1e-12)
    rb = Bg / jnp.maximum(float(_HW) - cntc, 1.0)
    rb = rb / jnp.maximum(
        jnp.sqrt(jnp.sum(rb * rb, axis=1, keepdims=True)), 1e-12)

    gt = lax.dot_general(rt, rt, (((1,), (1,)), ((), ())),
                         preferred_element_type=jnp.float32)
    gb = lax.dot_general(rt, rb, (((1,), (1,)), ((), ())),
                         preferred_element_type=jnp.float32)

    rowi = lax.broadcasted_iota(jnp.int32, (_N2, _N2), 0)
    colj = lax.broadcasted_iota(jnp.int32, (_N2, _N2), 1)
    tri = (rowi <= colj).astype(jnp.float32)

    cooc = (tidc != tidr) & (cntr != 0.0)
    cf = cooc.astype(jnp.float32)
    csum = lax.dot_general(cf, tri, (((1,), (0,)), ((), ())),
                           preferred_element_type=jnp.float32)
    ndiff = csum[:, _N2 - 1:_N2]
    jf = colj.astype(jnp.float32)
    key = jnp.where(cooc, csum - 1.0, ndiff + jf - csum)

    pcond = (tidc == tidr) & (colj != tidc)
    pf = pcond.astype(jnp.float32)
    psum = lax.dot_general(pf, tri, (((1,), (0,)), ((), ())),
                           preferred_element_type=jnp.float32)
    onehot = pf * (psum == 1.0).astype(jnp.float32)
    has_pos = psum[:, _N2 - 1:_N2] > 0.0
    fallback = (colj == 0).astype(jnp.float32)
    oh = jnp.where(has_pos, onehot, fallback)
    pos_logit = jnp.sum(oh * gt, axis=1, keepdims=True)

    cols = []
    for k in range(N_NEGATIVES):
        pk = P[:, k:k + 1].astype(jnp.float32)
        sel = pk < ndiff
        g = jnp.where(sel, gt, gb)
        match = (key == pk)
        cols.append(jnp.sum(jnp.where(match, g, 0.0), axis=1, keepdims=True))
    nl = jnp.concatenate(cols, axis=1) / TEMPERATURE
    pos = pos_logit / TEMPERATURE
    m = jnp.max(nl, axis=1, keepdims=True)
    row_loss = jnp.log(jnp.sum(jnp.exp(nl - m), axis=1, keepdims=True)) \
        - (pos - m)
    out_ref[...] = jnp.sum(row_loss, axis=0, keepdims=True) / float(_N2)


@functools.partial(jax.jit, static_argnames=())
def _run(features, labels, task_ids, perms):
    pool = pl.pallas_call(
        _pool_body,
        grid=(_B,),
        in_specs=[
            pl.BlockSpec((1, _D, 128, 128), lambda i: (i, 0, 0, 0)),
            pl.BlockSpec((1, 2, 128, 128), lambda i: (i, 0, 0, 0)),
        ],
        out_specs=[
            pl.BlockSpec((1, 2, _D), lambda i: (i, 0, 0)),
            pl.BlockSpec((1, 2, _D), lambda i: (i, 0, 0)),
            pl.BlockSpec((1, 2, _D), lambda i: (i, 0, 0)),
        ],
        out_shape=[
            jax.ShapeDtypeStruct((_B, 2, _D), jnp.float32),
            jax.ShapeDtypeStruct((_B, 2, _D), jnp.float32),
            jax.ShapeDtypeStruct((_B, 2, _D), jnp.float32),
        ],
    )
    t_sums, b_sums, cnts = pool(features, labels)

    t2 = t_sums.reshape(_N2, _D)
    b2 = b_sums.reshape(_N2, _D)
    cnt = cnts[:, :, 0].reshape(_N2)

    finale = pl.pallas_call(
        _finale_body,
        out_shape=jax.ShapeDtypeStruct((1, 1), jnp.float32),
    )
    loss = finale(
        t2, b2,
        cnt.reshape(_N2, 1), cnt.reshape(1, _N2),
        task_ids.reshape(_N2, 1), task_ids.reshape(1, _N2),
        perms,
    )
    return loss[0, 0]


@jax.jit
def _run_mix(features, labels, task_ids, perms):
    osc = _sc_split_pool(features, labels).reshape(_NSC, 6, _D, 16)
    pool = pl.pallas_call(
        _pool_body,
        grid=(_NTC,),
        in_specs=[
            pl.BlockSpec((1, _D, 128, 128), lambda i: (i, 0, 0, 0)),
            pl.BlockSpec((1, 2, 128, 128), lambda i: (i, 0, 0, 0)),
        ],
        out_specs=[
            pl.BlockSpec((1, 2, _D), lambda i: (i, 0, 0)),
            pl.BlockSpec((1, 2, _D), lambda i: (i, 0, 0)),
            pl.BlockSpec((1, 2, _D), lambda i: (i, 0, 0)),
        ],
        out_shape=[
            jax.ShapeDtypeStruct((_NTC, 2, _D), jnp.float32),
            jax.ShapeDtypeStruct((_NTC, 2, _D), jnp.float32),
            jax.ShapeDtypeStruct((_NTC, 2, _D), jnp.float32),
        ],
    )
    t_sums, b_sums, cnts = pool(features, labels)

    ttc = t_sums.reshape(2 * _NTC, _D)
    btc = b_sums.reshape(2 * _NTC, _D)
    ctc = cnts[:, :, 0].reshape(2 * _NTC)
    tsc = osc[:, 0:2].reshape(2 * _NSC, _D, 16)
    bsc = osc[:, 2:4].reshape(2 * _NSC, _D, 16)
    csc = osc[:, 4:6, 0, :].reshape(2 * _NSC, 16)

    finale = pl.pallas_call(
        _finale_mix_body,
        out_shape=jax.ShapeDtypeStruct((1, 1), jnp.float32),
    )
    loss = finale(
        ttc, btc, ctc.reshape(2 * _NTC, 1), ctc.reshape(1, 2 * _NTC),
        tsc, bsc, csc, csc.T,
        task_ids.reshape(_N2, 1), task_ids.reshape(1, _N2),
        perms,
    )
    return loss[0, 0]


@jax.jit
def _run_sc(features, labels, task_ids, perms):
    o = _sc_pool(features, labels).reshape(_B, 6, _D, 16)
    t2 = o[:, 0:2].reshape(_N2, _D, 16)
    b2 = o[:, 2:4].reshape(_N2, _D, 16)
    cntp = o[:, 4:6, 0, :].reshape(_N2, 16)
    finale = pl.pallas_call(
        _finale_sc_body,
        out_shape=jax.ShapeDtypeStruct((1, 1), jnp.float32),
    )
    loss = finale(
        t2, b2, cntp, cntp.T,
        task_ids.reshape(_N2, 1), task_ids.reshape(1, _N2),
        perms,
    )
    return loss[0, 0]


_rng = np.random.default_rng(0)
_PERMS = np.stack(
    [_rng.permutation(_D)[:N_NEGATIVES] for _ in range(_N2)]).astype(np.int32)


def kernel(features, labels, tasks):
    task_ids = jnp.stack([2 * tasks, 2 * tasks + 1], axis=1).reshape(-1)
    return _run_mix(features, labels, task_ids.astype(jnp.int32), _PERMS)


# speedup vs baseline: 2.8940x; 1.0037x over previous
import functools

import jax
import jax.numpy as jnp
import numpy as np
from jax import lax
from jax.experimental import pallas as pl
from jax.experimental.pallas import tpu as pltpu
from jax.experimental.pallas import tpu_sc as plsc

TEMPERATURE = 0.07
N_NEGATIVES = 32
_B = 64
_D = 64
_HW = 128 * 128
_N2 = 2 * _B


def _pool_body(f_ref, l_ref, t_ref, b_ref, c_ref):
    f = f_ref[0]
    l0 = l_ref[0, 0]
    l1 = l_ref[0, 1]
    t0p = jnp.sum(f * l0[None, :, :], axis=1)
    t1p = jnp.sum(f * l1[None, :, :], axis=1)
    sp = jnp.sum(f, axis=1)
    t0 = jnp.sum(t0p, axis=1)
    t1 = jnp.sum(t1p, axis=1)
    s = jnp.sum(sp, axis=1)
    t_ref[0, 0] = t0
    t_ref[0, 1] = t1
    b_ref[0, 0] = s - t0
    b_ref[0, 1] = s - t1
    c_ref[0, 0] = jnp.broadcast_to(jnp.sum(l0), (_D,))
    c_ref[0, 1] = jnp.broadcast_to(jnp.sum(l1), (_D,))


_NC = 2
_NS = 16
_DT = 8
_NSC = 16
_NTC = _B - _NSC
_UT = _D // _DT
_UPW = _NSC * _UT // (_NC * _NS)


def _sc_split_body(f_hbm, l_hbm, out_hbm, lab_v, fbuf, out_v, sem0, sem1):
    wid = lax.axis_index("s") * _NC + lax.axis_index("c")

    def do_unit(k, carry):
        unit = wid * _UPW + k
        bo = unit // _UT
        b = _NTC + bo
        dt = unit % _UT
        pltpu.sync_copy(l_hbm.at[b], lab_v)

        @pl.when(dt == 0)
        def _():
            def cnt_body(i, cc):
                c0, c1 = cc
                h = i >> 3
                w0 = (i & 7) * 16
                return (c0 + lab_v[0, h, pl.ds(w0, 16)],
                        c1 + lab_v[1, h, pl.ds(w0, 16)])

            c0v, c1v = plsc.parallel_loop(
                0, 1024, unroll=4,
                carry=(jnp.zeros((16,), jnp.float32),
                       jnp.zeros((16,), jnp.float32)))(cnt_body)
            out_v[pl.ds(4 * 16 * 16, 16)] = c0v
            out_v[pl.ds(4 * 16 * 16 + 16, 16)] = c1v
            pltpu.sync_copy(
                out_v.at[pl.ds(4 * 16 * 16, 16)],
                out_hbm.at[bo, pl.ds(4 * _D * 16, 16)])
            pltpu.sync_copy(
                out_v.at[pl.ds(4 * 16 * 16 + 16, 16)],
                out_hbm.at[bo, pl.ds(5 * _D * 16, 16)])

        sems = (sem0, sem1)
        copies = [None, None]
        copies[0] = pltpu.async_copy(
            f_hbm.at[b, pl.ds(dt * _DT, _DT), pl.ds(0, 16), :],
            fbuf.at[0], sem0)
        accs = tuple(jnp.zeros((16,), jnp.float32) for _ in range(3 * _DT))
        for hc in range(8):
            cur = hc % 2
            if hc < 7:
                nxt = (hc + 1) % 2
                copies[nxt] = pltpu.async_copy(
                    f_hbm.at[b, pl.ds(dt * _DT, _DT),
                             pl.ds((hc + 1) * 16, 16), :],
                    fbuf.at[nxt], sems[nxt])
            copies[cur].wait()

            def chunk_body(i, acc, _cur=cur, _hc=hc):
                h = i >> 3
                w0 = (i & 7) * 16
                l0c = lab_v[0, _hc * 16 + h, pl.ds(w0, 16)]
                l1c = lab_v[1, _hc * 16 + h, pl.ds(w0, 16)]
                t0s, t1s, ss = [], [], []
                for j in range(_DT):
                    fv = fbuf[_cur, j, h, pl.ds(w0, 16)]
                    t0s.append(acc[j] + fv * l0c)
                    t1s.append(acc[_DT + j] + fv * l1c)
                    ss.append(acc[2 * _DT + j] + fv)
                return tuple(t0s + t1s + ss)

            accs = plsc.parallel_loop(
                0, 128, unroll=4, carry=accs)(chunk_body)
        for j in range(_DT):
            out_v[pl.ds((0 * _DT + j) * 16, 16)] = accs[j]
            out_v[pl.ds((1 * _DT + j) * 16, 16)] = accs[_DT + j]
            out_v[pl.ds((2 * _DT + j) * 16, 16)] = \
                accs[2 * _DT + j] - accs[j]
            out_v[pl.ds((3 * _DT + j) * 16, 16)] = \
                accs[2 * _DT + j] - accs[_DT + j]
        for srow in range(4):
            pltpu.sync_copy(
                out_v.at[pl.ds(srow * _DT * 16, _DT * 16)],
                out_hbm.at[bo,
                           pl.ds((srow * _D + dt * _DT) * 16, _DT * 16)])
        return carry

    lax.fori_loop(0, _UPW, do_unit, 0)


_sc_split_pool = functools.partial(
    pl.kernel,
    out_type=jax.ShapeDtypeStruct((_NSC, 6 * _D * 16), jnp.float32),
    mesh=plsc.VectorSubcoreMesh(core_axis_name="c", subcore_axis_name="s"),
    scratch_types=[
        pltpu.VMEM((2, 128, 128), jnp.float32),
        pltpu.VMEM((2, _DT, 16, 128), jnp.float32),
        pltpu.VMEM((4 * 16 * 16 + 32,), jnp.float32),
        pltpu.SemaphoreType.DMA,
        pltpu.SemaphoreType.DMA,
    ],
)(_sc_split_body)


def _sc_pool_body(f_hbm, l_hbm, out_hbm, lab_v, fbuf, out_v, sem0, sem1):
    wid = lax.axis_index("s") * _NC + lax.axis_index("c")

    def do_image(img, carry):
        b = wid * 2 + img
        pltpu.sync_copy(l_hbm.at[b], lab_v)

        def cnt_body(i, cc):
            c0, c1 = cc
            h = i >> 3
            w0 = (i & 7) * 16
            return (c0 + lab_v[0, h, pl.ds(w0, 16)],
                    c1 + lab_v[1, h, pl.ds(w0, 16)])

        c0v, c1v = lax.fori_loop(
            0, 1024, cnt_body,
            (jnp.zeros((16,), jnp.float32), jnp.zeros((16,), jnp.float32)))
        out_v[pl.ds(4 * _D * 16, 16)] = c0v
        out_v[pl.ds(5 * _D * 16, 16)] = c1v

        sems = (sem0, sem1)
        for dt in range(_D // _DT):
            copies = [None, None]
            copies[0] = pltpu.async_copy(
                f_hbm.at[b, pl.ds(dt * _DT, _DT), pl.ds(0, 16), :],
                fbuf.at[0], sem0)
            accs = tuple(jnp.zeros((16,), jnp.float32) for _ in range(3 * _DT))
            for hc in range(8):
                cur = hc % 2
                if hc < 7:
                    nxt = (hc + 1) % 2
                    copies[nxt] = pltpu.async_copy(
                        f_hbm.at[b, pl.ds(dt * _DT, _DT),
                                 pl.ds((hc + 1) * 16, 16), :],
                        fbuf.at[nxt], sems[nxt])
                copies[cur].wait()

                def chunk_body(i, acc, _cur=cur, _hc=hc):
                    h = i >> 3
                    w0 = (i & 7) * 16
                    l0c = lab_v[0, _hc * 16 + h, pl.ds(w0, 16)]
                    l1c = lab_v[1, _hc * 16 + h, pl.ds(w0, 16)]
                    t0s, t1s, ss = [], [], []
                    for j in range(_DT):
                        fv = fbuf[_cur, j, h, pl.ds(w0, 16)]
                        t0s.append(acc[j] + fv * l0c)
                        t1s.append(acc[_DT + j] + fv * l1c)
                        ss.append(acc[2 * _DT + j] + fv)
                    return tuple(t0s + t1s + ss)

                accs = lax.fori_loop(0, 128, chunk_body, accs)
            for j in range(_DT):
                d = dt * _DT + j
                out_v[pl.ds((0 * _D + d) * 16, 16)] = accs[j]
                out_v[pl.ds((1 * _D + d) * 16, 16)] = accs[_DT + j]
                out_v[pl.ds((2 * _D + d) * 16, 16)] = \
                    accs[2 * _DT + j] - accs[j]
                out_v[pl.ds((3 * _D + d) * 16, 16)] = \
                    accs[2 * _DT + j] - accs[_DT + j]
        pltpu.sync_copy(out_v, out_hbm.at[b])
        return carry

    lax.fori_loop(0, 2, do_image, 0)


_sc_pool = functools.partial(
    pl.kernel,
    out_type=jax.ShapeDtypeStruct((_B, 6 * _D * 16), jnp.float32),
    mesh=plsc.VectorSubcoreMesh(core_axis_name="c", subcore_axis_name="s"),
    scratch_types=[
        pltpu.VMEM((2, 128, 128), jnp.float32),
        pltpu.VMEM((2, _DT, 16, 128), jnp.float32),
        pltpu.VMEM((6 * _D * 16,), jnp.float32),
        pltpu.SemaphoreType.DMA,
        pltpu.SemaphoreType.DMA,
    ],
)(_sc_pool_body)


def _finale_mix_body(ttc_ref, btc_ref, ctcc_ref, ctcr_ref, tsc_ref, bsc_ref,
                     cscp_ref, cscpt_ref, tidc_ref, tidr_ref, p_ref, out_ref):
    T = jnp.concatenate(
        [ttc_ref[...], jnp.sum(tsc_ref[...], axis=2)], axis=0)
    Bg = jnp.concatenate(
        [btc_ref[...], jnp.sum(bsc_ref[...], axis=2)], axis=0)
    cntc = jnp.concatenate(
        [ctcc_ref[...], jnp.sum(cscp_ref[...], axis=1, keepdims=True)],
        axis=0)
    cntr = jnp.concatenate(
        [ctcr_ref[...], jnp.sum(cscpt_ref[...], axis=0, keepdims=True)],
        axis=1)
    _finale_math(T, Bg, cntc, cntr, tidc_ref[...], tidr_ref[...], p_ref[...],
                 out_ref)


def _finale_sc_body(t_ref, b_ref, cpc_ref, cpr_ref, tidc_ref, tidr_ref,
                    p_ref, out_ref):
    T = jnp.sum(t_ref[...], axis=2)
    Bg = jnp.sum(b_ref[...], axis=2)
    cntc = jnp.sum(cpc_ref[...], axis=1, keepdims=True)
    cntr = jnp.sum(cpr_ref[...], axis=0, keepdims=True)
    _finale_math(T, Bg, cntc, cntr, tidc_ref[...], tidr_ref[...], p_ref[...],
                 out_ref)


def _finale_body(t_ref, b_ref, cc_ref, cr_ref, tidc_ref, tidr_ref, p_ref,
                 out_ref):
    _finale_math(t_ref[...], b_ref[...], cc_ref[...], cr_ref[...],
                 tidc_ref[...], tidr_ref[...], p_ref[...], out_ref)


def _finale_math(T, Bg, cntc, cntr, tidc, tidr, P, out_ref):
    rt = T / jnp.maximum(cntc, 1.0)
    rt = rt / jnp.maximum(
        jnp.sqrt(jnp.sum(rt * rt, axis=1, keepdims=True)), 1e-12)
    rb = Bg / jnp.maximum(float(_HW) - cntc, 1.0)
    rb = rb / jnp.maximum(
        jnp.sqrt(jnp.sum(rb * rb, axis=1, keepdims=True)), 1e-12)

    gt = lax.dot_general(rt, rt, (((1,), (1,)), ((), ())),
                         preferred_element_type=jnp.float32)
    gb = lax.dot_general(rt, rb, (((1,), (1,)), ((), ())),
                         preferred_element_type=jnp.float32)

    rowi = lax.broadcasted_iota(jnp.int32, (_N2, _N2), 0)
    colj = lax.broadcasted_iota(jnp.int32, (_N2, _N2), 1)
    tri = (rowi <= colj).astype(jnp.float32)

    cooc = (tidc != tidr) & (cntr != 0.0)
    cf = cooc.astype(jnp.float32)
    csum = lax.dot_general(cf, tri, (((1,), (0,)), ((), ())),
                           preferred_element_type=jnp.float32)
    ndiff = csum[:, _N2 - 1:_N2]
    jf = colj.astype(jnp.float32)
    key = jnp.where(cooc, csum - 1.0, ndiff + jf - csum)

    pcond = (tidc == tidr) & (colj != tidc)
    pf = pcond.astype(jnp.float32)
    psum = lax.dot_general(pf, tri, (((1,), (0,)), ((), ())),
                           preferred_element_type=jnp.float32)
    onehot = pf * (psum == 1.0).astype(jnp.float32)
    has_pos = psum[:, _N2 - 1:_N2] > 0.0
    fallback = (colj == 0).astype(jnp.float32)
    oh = jnp.where(has_pos, onehot, fallback)
    pos_logit = jnp.sum(oh * gt, axis=1, keepdims=True)

    cols = []
    for k in range(N_NEGATIVES):
        pk = P[:, k:k + 1].astype(jnp.float32)
        sel = pk < ndiff
        g = jnp.where(sel, gt, gb)
        match = (key == pk)
        cols.append(jnp.sum(jnp.where(match, g, 0.0), axis=1, keepdims=True))
    nl = jnp.concatenate(cols, axis=1) / TEMPERATURE
    pos = pos_logit / TEMPERATURE
    m = jnp.max(nl, axis=1, keepdims=True)
    row_loss = jnp.log(jnp.sum(jnp.exp(nl - m), axis=1, keepdims=True)) \
        - (pos - m)
    out_ref[...] = jnp.sum(row_loss, axis=0, keepdims=True) / float(_N2)


@functools.partial(jax.jit, static_argnames=())
def _run(features, labels, task_ids, perms):
    pool = pl.pallas_call(
        _pool_body,
        grid=(_B,),
        in_specs=[
            pl.BlockSpec((1, _D, 128, 128), lambda i: (i, 0, 0, 0)),
            pl.BlockSpec((1, 2, 128, 128), lambda i: (i, 0, 0, 0)),
        ],
        out_specs=[
            pl.BlockSpec((1, 2, _D), lambda i: (i, 0, 0)),
            pl.BlockSpec((1, 2, _D), lambda i: (i, 0, 0)),
            pl.BlockSpec((1, 2, _D), lambda i: (i, 0, 0)),
        ],
        out_shape=[
            jax.ShapeDtypeStruct((_B, 2, _D), jnp.float32),
            jax.ShapeDtypeStruct((_B, 2, _D), jnp.float32),
            jax.ShapeDtypeStruct((_B, 2, _D), jnp.float32),
        ],
    )
    t_sums, b_sums, cnts = pool(features, labels)

    t2 = t_sums.reshape(_N2, _D)
    b2 = b_sums.reshape(_N2, _D)
    cnt = cnts[:, :, 0].reshape(_N2)

    finale = pl.pallas_call(
        _finale_body,
        out_shape=jax.ShapeDtypeStruct((1, 1), jnp.float32),
    )
    loss = finale(
        t2, b2,
        cnt.reshape(_N2, 1), cnt.reshape(1, _N2),
        task_ids.reshape(_N2, 1), task_ids.reshape(1, _N2),
        perms,
    )
    return loss[0, 0]


@jax.jit
def _run_mix(features, labels, task_ids, perms):
    osc = _sc_split_pool(features, labels).reshape(_NSC, 6, _D, 16)
    pool = pl.pallas_call(
        _pool_body,
        grid=(_NTC,),
        in_specs=[
            pl.BlockSpec((1, _D, 128, 128), lambda i: (i, 0, 0, 0)),
            pl.BlockSpec((1, 2, 128, 128), lambda i: (i, 0, 0, 0)),
        ],
        out_specs=[
            pl.BlockSpec((1, 2, _D), lambda i: (i, 0, 0)),
            pl.BlockSpec((1, 2, _D), lambda i: (i, 0, 0)),
            pl.BlockSpec((1, 2, _D), lambda i: (i, 0, 0)),
        ],
        out_shape=[
            jax.ShapeDtypeStruct((_NTC, 2, _D), jnp.float32),
            jax.ShapeDtypeStruct((_NTC, 2, _D), jnp.float32),
            jax.ShapeDtypeStruct((_NTC, 2, _D), jnp.float32),
        ],
    )
    t_sums, b_sums, cnts = pool(features, labels)

    ttc = t_sums.reshape(2 * _NTC, _D)
    btc = b_sums.reshape(2 * _NTC, _D)
    ctc = cnts[:, :, 0].reshape(2 * _NTC)
    tsc = osc[:, 0:2].reshape(2 * _NSC, _D, 16)
    bsc = osc[:, 2:4].reshape(2 * _NSC, _D, 16)
    csc = osc[:, 4:6, 0, :].reshape(2 * _NSC, 16)

    finale = pl.pallas_call(
        _finale_mix_body,
        out_shape=jax.ShapeDtypeStruct((1, 1), jnp.float32),
    )
    loss = finale(
        ttc, btc, ctc.reshape(2 * _NTC, 1), ctc.reshape(1, 2 * _NTC),
        tsc, bsc, csc, csc.T,
        task_ids.reshape(_N2, 1), task_ids.reshape(1, _N2),
        perms,
    )
    return loss[0, 0]


@jax.jit
def _run_sc(features, labels, task_ids, perms):
    o = _sc_pool(features, labels).reshape(_B, 6, _D, 16)
    t2 = o[:, 0:2].reshape(_N2, _D, 16)
    b2 = o[:, 2:4].reshape(_N2, _D, 16)
    cntp = o[:, 4:6, 0, :].reshape(_N2, 16)
    finale = pl.pallas_call(
        _finale_sc_body,
        out_shape=jax.ShapeDtypeStruct((1, 1), jnp.float32),
    )
    loss = finale(
        t2, b2, cntp, cntp.T,
        task_ids.reshape(_N2, 1), task_ids.reshape(1, _N2),
        perms,
    )
    return loss[0, 0]


_rng = np.random.default_rng(0)
_PERMS = np.stack(
    [_rng.permutation(_D)[:N_NEGATIVES] for _ in range(_N2)]).astype(np.int32)


def kernel(features, labels, tasks):
    task_ids = jnp.stack([2 * tasks, 2 * tasks + 1], axis=1).reshape(-1)
    return _run_mix(features, labels, task_ids.astype(jnp.int32), _PERMS)


# speedup vs baseline: 2.9140x; 1.0069x over previous
import functools

import jax
import jax.numpy as jnp
import numpy as np
from jax import lax
from jax.experimental import pallas as pl
from jax.experimental.pallas import tpu as pltpu
from jax.experimental.pallas import tpu_sc as plsc

TEMPERATURE = 0.07
N_NEGATIVES = 32
_B = 64
_D = 64
_HW = 128 * 128
_N2 = 2 * _B


def _pool_body(f_ref, l_ref, t_ref, b_ref, c_ref):
    f = f_ref[0]
    l0 = l_ref[0, 0]
    l1 = l_ref[0, 1]
    t0p = jnp.sum(f * l0[None, :, :], axis=1)
    t1p = jnp.sum(f * l1[None, :, :], axis=1)
    sp = jnp.sum(f, axis=1)
    t0 = jnp.sum(t0p, axis=1)
    t1 = jnp.sum(t1p, axis=1)
    s = jnp.sum(sp, axis=1)
    t_ref[0, 0] = t0
    t_ref[0, 1] = t1
    b_ref[0, 0] = s - t0
    b_ref[0, 1] = s - t1
    c_ref[0, 0] = jnp.broadcast_to(jnp.sum(l0), (_D,))
    c_ref[0, 1] = jnp.broadcast_to(jnp.sum(l1), (_D,))


_NC = 2
_NS = 16
_DT = 8
_NSC = 16
_NTC = _B - _NSC
_UT = _D // _DT
_UPW = _NSC * _UT // (_NC * _NS)


def _sc_split_body(f_hbm, l_hbm, t_hbm, b_hbm, c_hbm, lab_v, fbuf, out_v,
                   sem0, sem1):
    wid = lax.axis_index("s") * _NC + lax.axis_index("c")

    def do_unit(k, carry):
        unit = wid * _UPW + k
        bo = unit // _UT
        b = _NTC + bo
        dt = unit % _UT
        pltpu.sync_copy(l_hbm.at[b], lab_v)

        @pl.when(dt == 0)
        def _():
            def cnt_body(i, cc):
                c0, c1 = cc
                h = i >> 3
                w0 = (i & 7) * 16
                return (c0 + lab_v[0, h, pl.ds(w0, 16)],
                        c1 + lab_v[1, h, pl.ds(w0, 16)])

            c0v, c1v = plsc.parallel_loop(
                0, 1024, unroll=4,
                carry=(jnp.zeros((16,), jnp.float32),
                       jnp.zeros((16,), jnp.float32)))(cnt_body)
            out_v[pl.ds(4 * 16 * 16, 16)] = c0v
            out_v[pl.ds(4 * 16 * 16 + 16, 16)] = c1v
            pltpu.sync_copy(
                out_v.at[pl.ds(4 * 16 * 16, 16)],
                c_hbm.at[pl.ds(2 * bo * 16, 16)])
            pltpu.sync_copy(
                out_v.at[pl.ds(4 * 16 * 16 + 16, 16)],
                c_hbm.at[pl.ds((2 * bo + 1) * 16, 16)])

        sems = (sem0, sem1)
        copies = [None, None]
        copies[0] = pltpu.async_copy(
            f_hbm.at[b, pl.ds(dt * _DT, _DT), pl.ds(0, 16), :],
            fbuf.at[0], sem0)
        accs = tuple(jnp.zeros((16,), jnp.float32) for _ in range(3 * _DT))
        for hc in range(8):
            cur = hc % 2
            if hc < 7:
                nxt = (hc + 1) % 2
                copies[nxt] = pltpu.async_copy(
                    f_hbm.at[b, pl.ds(dt * _DT, _DT),
                             pl.ds((hc + 1) * 16, 16), :],
                    fbuf.at[nxt], sems[nxt])
            copies[cur].wait()

            def chunk_body(i, acc, _cur=cur, _hc=hc):
                h = i >> 3
                w0 = (i & 7) * 16
                l0c = lab_v[0, _hc * 16 + h, pl.ds(w0, 16)]
                l1c = lab_v[1, _hc * 16 + h, pl.ds(w0, 16)]
                t0s, t1s, ss = [], [], []
                for j in range(_DT):
                    fv = fbuf[_cur, j, h, pl.ds(w0, 16)]
                    t0s.append(acc[j] + fv * l0c)
                    t1s.append(acc[_DT + j] + fv * l1c)
                    ss.append(acc[2 * _DT + j] + fv)
                return tuple(t0s + t1s + ss)

            accs = plsc.parallel_loop(
                0, 128, unroll=4, carry=accs)(chunk_body)
        for j in range(_DT):
            out_v[pl.ds((0 * _DT + j) * 16, 16)] = accs[j]
            out_v[pl.ds((1 * _DT + j) * 16, 16)] = accs[_DT + j]
            out_v[pl.ds((2 * _DT + j) * 16, 16)] = \
                accs[2 * _DT + j] - accs[j]
            out_v[pl.ds((3 * _DT + j) * 16, 16)] = \
                accs[2 * _DT + j] - accs[_DT + j]
        for srow, dst in ((0, t_hbm), (1, t_hbm), (2, b_hbm), (3, b_hbm)):
            pltpu.sync_copy(
                out_v.at[pl.ds(srow * _DT * 16, _DT * 16)],
                dst.at[pl.ds((2 * bo + (srow % 2)) * _D * 16 + dt * _DT * 16,
                             _DT * 16)])
        return carry

    lax.fori_loop(0, _UPW, do_unit, 0)


_sc_split_pool = functools.partial(
    pl.kernel,
    out_type=[
        jax.ShapeDtypeStruct((2 * _NSC * _D * 16,), jnp.float32),
        jax.ShapeDtypeStruct((2 * _NSC * _D * 16,), jnp.float32),
        jax.ShapeDtypeStruct((2 * _NSC * 16,), jnp.float32),
    ],
    mesh=plsc.VectorSubcoreMesh(core_axis_name="c", subcore_axis_name="s"),
    scratch_types=[
        pltpu.VMEM((2, 128, 128), jnp.float32),
        pltpu.VMEM((2, _DT, 16, 128), jnp.float32),
        pltpu.VMEM((4 * 16 * 16 + 32,), jnp.float32),
        pltpu.SemaphoreType.DMA,
        pltpu.SemaphoreType.DMA,
    ],
)(_sc_split_body)


def _sc_pool_body(f_hbm, l_hbm, out_hbm, lab_v, fbuf, out_v, sem0, sem1):
    wid = lax.axis_index("s") * _NC + lax.axis_index("c")

    def do_image(img, carry):
        b = wid * 2 + img
        pltpu.sync_copy(l_hbm.at[b], lab_v)

        def cnt_body(i, cc):
            c0, c1 = cc
            h = i >> 3
            w0 = (i & 7) * 16
            return (c0 + lab_v[0, h, pl.ds(w0, 16)],
                    c1 + lab_v[1, h, pl.ds(w0, 16)])

        c0v, c1v = lax.fori_loop(
            0, 1024, cnt_body,
            (jnp.zeros((16,), jnp.float32), jnp.zeros((16,), jnp.float32)))
        out_v[pl.ds(4 * _D * 16, 16)] = c0v
        out_v[pl.ds(5 * _D * 16, 16)] = c1v

        sems = (sem0, sem1)
        for dt in range(_D // _DT):
            copies = [None, None]
            copies[0] = pltpu.async_copy(
                f_hbm.at[b, pl.ds(dt * _DT, _DT), pl.ds(0, 16), :],
                fbuf.at[0], sem0)
            accs = tuple(jnp.zeros((16,), jnp.float32) for _ in range(3 * _DT))
            for hc in range(8):
                cur = hc % 2
                if hc < 7:
                    nxt = (hc + 1) % 2
                    copies[nxt] = pltpu.async_copy(
                        f_hbm.at[b, pl.ds(dt * _DT, _DT),
                                 pl.ds((hc + 1) * 16, 16), :],
                        fbuf.at[nxt], sems[nxt])
                copies[cur].wait()

                def chunk_body(i, acc, _cur=cur, _hc=hc):
                    h = i >> 3
                    w0 = (i & 7) * 16
                    l0c = lab_v[0, _hc * 16 + h, pl.ds(w0, 16)]
                    l1c = lab_v[1, _hc * 16 + h, pl.ds(w0, 16)]
                    t0s, t1s, ss = [], [], []
                    for j in range(_DT):
                        fv = fbuf[_cur, j, h, pl.ds(w0, 16)]
                        t0s.append(acc[j] + fv * l0c)
                        t1s.append(acc[_DT + j] + fv * l1c)
                        ss.append(acc[2 * _DT + j] + fv)
                    return tuple(t0s + t1s + ss)

                accs = lax.fori_loop(0, 128, chunk_body, accs)
            for j in range(_DT):
                d = dt * _DT + j
                out_v[pl.ds((0 * _D + d) * 16, 16)] = accs[j]
                out_v[pl.ds((1 * _D + d) * 16, 16)] = accs[_DT + j]
                out_v[pl.ds((2 * _D + d) * 16, 16)] = \
                    accs[2 * _DT + j] - accs[j]
                out_v[pl.ds((3 * _D + d) * 16, 16)] = \
                    accs[2 * _DT + j] - accs[_DT + j]
        pltpu.sync_copy(out_v, out_hbm.at[b])
        return carry

    lax.fori_loop(0, 2, do_image, 0)


_sc_pool = functools.partial(
    pl.kernel,
    out_type=jax.ShapeDtypeStruct((_B, 6 * _D * 16), jnp.float32),
    mesh=plsc.VectorSubcoreMesh(core_axis_name="c", subcore_axis_name="s"),
    scratch_types=[
        pltpu.VMEM((2, 128, 128), jnp.float32),
        pltpu.VMEM((2, _DT, 16, 128), jnp.float32),
        pltpu.VMEM((6 * _D * 16,), jnp.float32),
        pltpu.SemaphoreType.DMA,
        pltpu.SemaphoreType.DMA,
    ],
)(_sc_pool_body)


def _finale_mix_body(ttc_ref, btc_ref, ctcc_ref, ctcr_ref, tsc_ref, bsc_ref,
                     cscp_ref, cscpt_ref, tidc_ref, tidr_ref, p_ref, out_ref):
    gr = lax.broadcasted_iota(jnp.int32, (_D * 16, _D), 0)
    gc = lax.broadcasted_iota(jnp.int32, (_D * 16, _D), 1)
    M = ((gr >> 4) == gc).astype(jnp.float32)
    tsc = lax.dot_general(tsc_ref[...], M, (((1,), (0,)), ((), ())),
                          preferred_element_type=jnp.float32)
    bsc = lax.dot_general(bsc_ref[...], M, (((1,), (0,)), ((), ())),
                          preferred_element_type=jnp.float32)
    T = jnp.concatenate([ttc_ref[...], tsc], axis=0)
    Bg = jnp.concatenate([btc_ref[...], bsc], axis=0)
    cntc = jnp.concatenate(
        [ctcc_ref[...], jnp.sum(cscp_ref[...], axis=1, keepdims=True)],
        axis=0)
    cntr = jnp.concatenate(
        [ctcr_ref[...], jnp.sum(cscpt_ref[...], axis=0, keepdims=True)],
        axis=1)
    _finale_math(T, Bg, cntc, cntr, tidc_ref[...], tidr_ref[...], p_ref[...],
                 out_ref)


def _finale_sc_body(t_ref, b_ref, cpc_ref, cpr_ref, tidc_ref, tidr_ref,
                    p_ref, out_ref):
    T = jnp.sum(t_ref[...], axis=2)
    Bg = jnp.sum(b_ref[...], axis=2)
    cntc = jnp.sum(cpc_ref[...], axis=1, keepdims=True)
    cntr = jnp.sum(cpr_ref[...], axis=0, keepdims=True)
    _finale_math(T, Bg, cntc, cntr, tidc_ref[...], tidr_ref[...], p_ref[...],
                 out_ref)


def _finale_body(t_ref, b_ref, cc_ref, cr_ref, tidc_ref, tidr_ref, p_ref,
                 out_ref):
    _finale_math(t_ref[...], b_ref[...], cc_ref[...], cr_ref[...],
                 tidc_ref[...], tidr_ref[...], p_ref[...], out_ref)


def _finale_math(T, Bg, cntc, cntr, tidc, tidr, P, out_ref):
    rt = T / jnp.maximum(cntc, 1.0)
    rt = rt / jnp.maximum(
        jnp.sqrt(jnp.sum(rt * rt, axis=1, keepdims=True)), 1e-12)
    rb = Bg / jnp.maximum(float(_HW) - cntc, 1.0)
    rb = rb / jnp.maximum(
        jnp.sqrt(jnp.sum(rb * rb, axis=1, keepdims=True)), 1e-12)

    gt = lax.dot_general(rt, rt, (((1,), (1,)), ((), ())),
                         preferred_element_type=jnp.float32)
    gb = lax.dot_general(rt, rb, (((1,), (1,)), ((), ())),
                         preferred_element_type=jnp.float32)

    rowi = lax.broadcasted_iota(jnp.int32, (_N2, _N2), 0)
    colj = lax.broadcasted_iota(jnp.int32, (_N2, _N2), 1)
    tri = (rowi <= colj).astype(jnp.float32)

    cooc = (tidc != tidr) & (cntr != 0.0)
    cf = cooc.astype(jnp.float32)
    csum = lax.dot_general(cf, tri, (((1,), (0,)), ((), ())),
                           preferred_element_type=jnp.float32)
    ndiff = csum[:, _N2 - 1:_N2]
    jf = colj.astype(jnp.float32)
    key = jnp.where(cooc, csum - 1.0, ndiff + jf - csum)

    pcond = (tidc == tidr) & (colj != tidc)
    pf = pcond.astype(jnp.float32)
    psum = lax.dot_general(pf, tri, (((1,), (0,)), ((), ())),
                           preferred_element_type=jnp.float32)
    onehot = pf * (psum == 1.0).astype(jnp.float32)
    has_pos = psum[:, _N2 - 1:_N2] > 0.0
    fallback = (colj == 0).astype(jnp.float32)
    oh = jnp.where(has_pos, onehot, fallback)
    pos_logit = jnp.sum(oh * gt, axis=1, keepdims=True)

    cols = []
    for k in range(N_NEGATIVES):
        pk = P[:, k:k + 1].astype(jnp.float32)
        sel = pk < ndiff
        g = jnp.where(sel, gt, gb)
        match = (key == pk)
        cols.append(jnp.sum(jnp.where(match, g, 0.0), axis=1, keepdims=True))
    nl = jnp.concatenate(cols, axis=1) / TEMPERATURE
    pos = pos_logit / TEMPERATURE
    m = jnp.max(nl, axis=1, keepdims=True)
    row_loss = jnp.log(jnp.sum(jnp.exp(nl - m), axis=1, keepdims=True)) \
        - (pos - m)
    out_ref[...] = jnp.sum(row_loss, axis=0, keepdims=True) / float(_N2)


@functools.partial(jax.jit, static_argnames=())
def _run(features, labels, task_ids, perms):
    pool = pl.pallas_call(
        _pool_body,
        grid=(_B,),
        in_specs=[
            pl.BlockSpec((1, _D, 128, 128), lambda i: (i, 0, 0, 0)),
            pl.BlockSpec((1, 2, 128, 128), lambda i: (i, 0, 0, 0)),
        ],
        out_specs=[
            pl.BlockSpec((1, 2, _D), lambda i: (i, 0, 0)),
            pl.BlockSpec((1, 2, _D), lambda i: (i, 0, 0)),
            pl.BlockSpec((1, 2, _D), lambda i: (i, 0, 0)),
        ],
        out_shape=[
            jax.ShapeDtypeStruct((_B, 2, _D), jnp.float32),
            jax.ShapeDtypeStruct((_B, 2, _D), jnp.float32),
            jax.ShapeDtypeStruct((_B, 2, _D), jnp.float32),
        ],
    )
    t_sums, b_sums, cnts = pool(features, labels)

    t2 = t_sums.reshape(_N2, _D)
    b2 = b_sums.reshape(_N2, _D)
    cnt = cnts[:, :, 0].reshape(_N2)

    finale = pl.pallas_call(
        _finale_body,
        out_shape=jax.ShapeDtypeStruct((1, 1), jnp.float32),
    )
    loss = finale(
        t2, b2,
        cnt.reshape(_N2, 1), cnt.reshape(1, _N2),
        task_ids.reshape(_N2, 1), task_ids.reshape(1, _N2),
        perms,
    )
    return loss[0, 0]


@jax.jit
def _run_mix(features, labels, task_ids, perms):
    tsc_f, bsc_f, csc_f = _sc_split_pool(features, labels)
    tsc = tsc_f.reshape(2 * _NSC, _D * 16)
    bsc = bsc_f.reshape(2 * _NSC, _D * 16)
    csc = csc_f.reshape(2 * _NSC, 16)
    pool = pl.pallas_call(
        _pool_body,
        grid=(_NTC,),
        in_specs=[
            pl.BlockSpec((1, _D, 128, 128), lambda i: (i, 0, 0, 0)),
            pl.BlockSpec((1, 2, 128, 128), lambda i: (i, 0, 0, 0)),
        ],
        out_specs=[
            pl.BlockSpec((1, 2, _D), lambda i: (i, 0, 0)),
            pl.BlockSpec((1, 2, _D), lambda i: (i, 0, 0)),
            pl.BlockSpec((1, 2, _D), lambda i: (i, 0, 0)),
        ],
        out_shape=[
            jax.ShapeDtypeStruct((_NTC, 2, _D), jnp.float32),
            jax.ShapeDtypeStruct((_NTC, 2, _D), jnp.float32),
            jax.ShapeDtypeStruct((_NTC, 2, _D), jnp.float32),
        ],
    )
    t_sums, b_sums, cnts = pool(features, labels)

    ttc = t_sums.reshape(2 * _NTC, _D)
    btc = b_sums.reshape(2 * _NTC, _D)
    ctc = cnts[:, :, 0].reshape(2 * _NTC)

    finale = pl.pallas_call(
        _finale_mix_body,
        out_shape=jax.ShapeDtypeStruct((1, 1), jnp.float32),
    )
    loss = finale(
        ttc, btc, ctc.reshape(2 * _NTC, 1), ctc.reshape(1, 2 * _NTC),
        tsc, bsc, csc, csc.T,
        task_ids.reshape(_N2, 1), task_ids.reshape(1, _N2),
        perms,
    )
    return loss[0, 0]


@jax.jit
def _run_sc(features, labels, task_ids, perms):
    o = _sc_pool(features, labels).reshape(_B, 6, _D, 16)
    t2 = o[:, 0:2].reshape(_N2, _D, 16)
    b2 = o[:, 2:4].reshape(_N2, _D, 16)
    cntp = o[:, 4:6, 0, :].reshape(_N2, 16)
    finale = pl.pallas_call(
        _finale_sc_body,
        out_shape=jax.ShapeDtypeStruct((1, 1), jnp.float32),
    )
    loss = finale(
        t2, b2, cntp, cntp.T,
        task_ids.reshape(_N2, 1), task_ids.reshape(1, _N2),
        perms,
    )
    return loss[0, 0]


_rng = np.random.default_rng(0)
_PERMS = np.stack(
    [_rng.permutation(_D)[:N_NEGATIVES] for _ in range(_N2)]).astype(np.int32)


def kernel(features, labels, tasks):
    task_ids = jnp.stack([2 * tasks, 2 * tasks + 1], axis=1).reshape(-1)
    return _run_mix(features, labels, task_ids.astype(jnp.int32), _PERMS)


# speedup vs baseline: 3.0140x; 1.0343x over previous
import functools

import jax
import jax.numpy as jnp
import numpy as np
from jax import lax
from jax.experimental import pallas as pl
from jax.experimental.pallas import tpu as pltpu
from jax.experimental.pallas import tpu_sc as plsc

TEMPERATURE = 0.07
N_NEGATIVES = 32
_B = 64
_D = 64
_HW = 128 * 128
_N2 = 2 * _B


def _pool_body(f_ref, l_ref, t_ref, b_ref, c_ref):
    f = f_ref[0]
    l0 = l_ref[0, 0]
    l1 = l_ref[0, 1]
    t0p = jnp.sum(f * l0[None, :, :], axis=1)
    t1p = jnp.sum(f * l1[None, :, :], axis=1)
    sp = jnp.sum(f, axis=1)
    t0 = jnp.sum(t0p, axis=1)
    t1 = jnp.sum(t1p, axis=1)
    s = jnp.sum(sp, axis=1)
    t_ref[0, 0] = t0
    t_ref[0, 1] = t1
    b_ref[0, 0] = s - t0
    b_ref[0, 1] = s - t1
    c_ref[0, 0] = jnp.broadcast_to(jnp.sum(l0), (_D,))
    c_ref[0, 1] = jnp.broadcast_to(jnp.sum(l1), (_D,))


_NC = 2
_NS = 16
_DT = 8
_NSC = 16
_NTC = _B - _NSC
_UT = _D // _DT
_UPW = _NSC * _UT // (_NC * _NS)


def _sc_split_body(f_hbm, l_hbm, t_hbm, b_hbm, c_hbm, lab_v, fbuf, out_v,
                   sem0, sem1):
    wid = lax.axis_index("s") * _NC + lax.axis_index("c")

    def do_unit(k, carry):
        unit = wid * _UPW + k
        bo = unit // _UT
        b = _NTC + bo
        dt = unit % _UT
        pltpu.sync_copy(l_hbm.at[b], lab_v)

        @pl.when(dt == 0)
        def _():
            def cnt_body(i, cc):
                c0, c1 = cc
                h = i >> 3
                w0 = (i & 7) * 16
                return (c0 + lab_v[0, h, pl.ds(w0, 16)],
                        c1 + lab_v[1, h, pl.ds(w0, 16)])

            c0v, c1v = plsc.parallel_loop(
                0, 1024, unroll=4,
                carry=(jnp.zeros((16,), jnp.float32),
                       jnp.zeros((16,), jnp.float32)))(cnt_body)
            out_v[pl.ds(4 * 16 * 16, 16)] = c0v
            out_v[pl.ds(4 * 16 * 16 + 16, 16)] = c1v
            pltpu.sync_copy(
                out_v.at[pl.ds(4 * 16 * 16, 16)],
                c_hbm.at[pl.ds(2 * bo * 16, 16)])
            pltpu.sync_copy(
                out_v.at[pl.ds(4 * 16 * 16 + 16, 16)],
                c_hbm.at[pl.ds((2 * bo + 1) * 16, 16)])

        sems = (sem0, sem1)
        copies = [None, None]
        copies[0] = pltpu.async_copy(
            f_hbm.at[b, pl.ds(dt * _DT, _DT), pl.ds(0, 16), :],
            fbuf.at[0], sem0)
        accs = tuple(jnp.zeros((16,), jnp.float32) for _ in range(3 * _DT))
        for hc in range(8):
            cur = hc % 2
            if hc < 7:
                nxt = (hc + 1) % 2
                copies[nxt] = pltpu.async_copy(
                    f_hbm.at[b, pl.ds(dt * _DT, _DT),
                             pl.ds((hc + 1) * 16, 16), :],
                    fbuf.at[nxt], sems[nxt])
            copies[cur].wait()

            def chunk_body(i, acc, _cur=cur, _hc=hc):
                h = i >> 3
                w0 = (i & 7) * 16
                l0c = lab_v[0, _hc * 16 + h, pl.ds(w0, 16)]
                l1c = lab_v[1, _hc * 16 + h, pl.ds(w0, 16)]
                t0s, t1s, ss = [], [], []
                for j in range(_DT):
                    fv = fbuf[_cur, j, h, pl.ds(w0, 16)]
                    t0s.append(acc[j] + fv * l0c)
                    t1s.append(acc[_DT + j] + fv * l1c)
                    ss.append(acc[2 * _DT + j] + fv)
                return tuple(t0s + t1s + ss)

            accs = plsc.parallel_loop(
                0, 128, unroll=4, carry=accs)(chunk_body)
        for j in range(_DT):
            out_v[pl.ds((0 * _DT + j) * 16, 16)] = accs[j]
            out_v[pl.ds((1 * _DT + j) * 16, 16)] = accs[_DT + j]
            out_v[pl.ds((2 * _DT + j) * 16, 16)] = \
                accs[2 * _DT + j] - accs[j]
            out_v[pl.ds((3 * _DT + j) * 16, 16)] = \
                accs[2 * _DT + j] - accs[_DT + j]
        for srow, dst in ((0, t_hbm), (1, t_hbm), (2, b_hbm), (3, b_hbm)):
            pltpu.sync_copy(
                out_v.at[pl.ds(srow * _DT * 16, _DT * 16)],
                dst.at[pl.ds((2 * bo + (srow % 2)) * _D * 16 + dt * _DT * 16,
                             _DT * 16)])
        return carry

    lax.fori_loop(0, _UPW, do_unit, 0)


@functools.lru_cache(maxsize=None)
def _sc_split_pool_fn():
    return functools.partial(
        pl.kernel,
        out_type=[
            jax.ShapeDtypeStruct((2 * _NSC * _D * 16,), jnp.float32),
            jax.ShapeDtypeStruct((2 * _NSC * _D * 16,), jnp.float32),
            jax.ShapeDtypeStruct((2 * _NSC * 16,), jnp.float32),
        ],
        mesh=plsc.VectorSubcoreMesh(core_axis_name="c",
                                    subcore_axis_name="s"),
        scratch_types=[
            pltpu.VMEM((2, 128, 128), jnp.float32),
            pltpu.VMEM((2, _DT, 16, 128), jnp.float32),
            pltpu.VMEM((4 * 16 * 16 + 32,), jnp.float32),
            pltpu.SemaphoreType.DMA,
            pltpu.SemaphoreType.DMA,
        ],
    )(_sc_split_body)


def _finale_mix_body(ttc_ref, btc_ref, ctcc_ref, ctcr_ref, tsc_ref, bsc_ref,
                     cscp_ref, cscpt_ref, tidc_ref, tidr_ref, p_ref, out_ref):
    gr = lax.broadcasted_iota(jnp.int32, (_D * 16, _D), 0)
    gc = lax.broadcasted_iota(jnp.int32, (_D * 16, _D), 1)
    M = ((gr >> 4) == gc).astype(jnp.float32)
    tsc = lax.dot_general(tsc_ref[...], M, (((1,), (0,)), ((), ())),
                          preferred_element_type=jnp.float32)
    bsc = lax.dot_general(bsc_ref[...], M, (((1,), (0,)), ((), ())),
                          preferred_element_type=jnp.float32)
    T = jnp.concatenate([ttc_ref[...], tsc], axis=0)
    Bg = jnp.concatenate([btc_ref[...], bsc], axis=0)
    cntc = jnp.concatenate(
        [ctcc_ref[...], jnp.sum(cscp_ref[...], axis=1, keepdims=True)],
        axis=0)
    cntr = jnp.concatenate(
        [ctcr_ref[...], jnp.sum(cscpt_ref[...], axis=0, keepdims=True)],
        axis=1)
    _finale_math(T, Bg, cntc, cntr, tidc_ref[...], tidr_ref[...], p_ref[...],
                 out_ref)


def _finale_body(t_ref, b_ref, cc_ref, cr_ref, tidc_ref, tidr_ref, p_ref,
                 out_ref):
    _finale_math(t_ref[...], b_ref[...], cc_ref[...], cr_ref[...],
                 tidc_ref[...], tidr_ref[...], p_ref[...], out_ref)


def _finale_math(T, Bg, cntc, cntr, tidc, tidr, P, out_ref):
    rt = T / jnp.maximum(cntc, 1.0)
    rt = rt / jnp.maximum(
        jnp.sqrt(jnp.sum(rt * rt, axis=1, keepdims=True)), 1e-12)
    rb = Bg / jnp.maximum(float(_HW) - cntc, 1.0)
    rb = rb / jnp.maximum(
        jnp.sqrt(jnp.sum(rb * rb, axis=1, keepdims=True)), 1e-12)

    gt = lax.dot_general(rt, rt, (((1,), (1,)), ((), ())),
                         preferred_element_type=jnp.float32)
    gb = lax.dot_general(rt, rb, (((1,), (1,)), ((), ())),
                         preferred_element_type=jnp.float32)

    rowi = lax.broadcasted_iota(jnp.int32, (_N2, _N2), 0)
    colj = lax.broadcasted_iota(jnp.int32, (_N2, _N2), 1)
    tri = (rowi <= colj).astype(jnp.float32)

    cooc = (tidc != tidr) & (cntr != 0.0)
    cf = cooc.astype(jnp.float32)
    csum = lax.dot_general(cf, tri, (((1,), (0,)), ((), ())),
                           preferred_element_type=jnp.float32)
    ndiff = csum[:, _N2 - 1:_N2]
    jf = colj.astype(jnp.float32)
    key = jnp.where(cooc, csum - 1.0, ndiff + jf - csum)

    pcond = (tidc == tidr) & (colj != tidc)
    pf = pcond.astype(jnp.float32)
    psum = lax.dot_general(pf, tri, (((1,), (0,)), ((), ())),
                           preferred_element_type=jnp.float32)
    onehot = pf * (psum == 1.0).astype(jnp.float32)
    has_pos = psum[:, _N2 - 1:_N2] > 0.0
    fallback = (colj == 0).astype(jnp.float32)
    oh = jnp.where(has_pos, onehot, fallback)
    pos_logit = jnp.sum(oh * gt, axis=1, keepdims=True)

    nw = N_NEGATIVES * _N2
    qk = lax.broadcasted_iota(jnp.int32, (N_NEGATIVES, nw), 0)
    qq = lax.broadcasted_iota(jnp.int32, (N_NEGATIVES, nw), 1)
    ek = (qk == (qq // _N2)).astype(jnp.float32)
    fj = lax.broadcasted_iota(jnp.int32, (_N2, nw), 0)
    fq = lax.broadcasted_iota(jnp.int32, (_N2, nw), 1)
    fjm = (fj == (fq % _N2)).astype(jnp.float32)
    dots = functools.partial(lax.dot_general,
                             dimension_numbers=(((1,), (0,)), ((), ())),
                             preferred_element_type=jnp.float32)
    pf = P.astype(jnp.float32)
    selk = (pf < ndiff).astype(jnp.float32)
    pk_w = dots(pf, ek)
    sel_w = dots(selk, ek)
    key_w = dots(key, fjm)
    gt_w = dots(gt, fjm)
    gb_w = dots(gb, fjm)
    mw = jnp.where(key_w == pk_w,
                   jnp.where(sel_w > 0.5, gt_w, gb_w), 0.0)
    rk = lax.broadcasted_iota(jnp.int32, (nw, N_NEGATIVES), 0)
    rc = lax.broadcasted_iota(jnp.int32, (nw, N_NEGATIVES), 1)
    e2 = ((rk // _N2) == rc).astype(jnp.float32)
    nl = dots(mw, e2) / TEMPERATURE
    pos = pos_logit / TEMPERATURE
    m = jnp.max(nl, axis=1, keepdims=True)
    row_loss = jnp.log(jnp.sum(jnp.exp(nl - m), axis=1, keepdims=True)) \
        - (pos - m)
    out_ref[...] = jnp.sum(row_loss, axis=0, keepdims=True) / float(_N2)


@functools.partial(jax.jit, static_argnames=())
def _run(features, labels, task_ids, perms):
    pool = pl.pallas_call(
        _pool_body,
        grid=(_B,),
        in_specs=[
            pl.BlockSpec((1, _D, 128, 128), lambda i: (i, 0, 0, 0)),
            pl.BlockSpec((1, 2, 128, 128), lambda i: (i, 0, 0, 0)),
        ],
        out_specs=[
            pl.BlockSpec((1, 2, _D), lambda i: (i, 0, 0)),
            pl.BlockSpec((1, 2, _D), lambda i: (i, 0, 0)),
            pl.BlockSpec((1, 2, _D), lambda i: (i, 0, 0)),
        ],
        out_shape=[
            jax.ShapeDtypeStruct((_B, 2, _D), jnp.float32),
            jax.ShapeDtypeStruct((_B, 2, _D), jnp.float32),
            jax.ShapeDtypeStruct((_B, 2, _D), jnp.float32),
        ],
    )
    t_sums, b_sums, cnts = pool(features, labels)

    t2 = t_sums.reshape(_N2, _D)
    b2 = b_sums.reshape(_N2, _D)
    cnt = cnts[:, :, 0].reshape(_N2)

    finale = pl.pallas_call(
        _finale_body,
        out_shape=jax.ShapeDtypeStruct((1, 1), jnp.float32),
    )
    loss = finale(
        t2, b2,
        cnt.reshape(_N2, 1), cnt.reshape(1, _N2),
        task_ids.reshape(_N2, 1), task_ids.reshape(1, _N2),
        perms,
    )
    return loss[0, 0]


@jax.jit
def _run_mix(features, labels, task_ids, perms):
    tsc_f, bsc_f, csc_f = _sc_split_pool_fn()(features, labels)
    tsc = tsc_f.reshape(2 * _NSC, _D * 16)
    bsc = bsc_f.reshape(2 * _NSC, _D * 16)
    csc = csc_f.reshape(2 * _NSC, 16)
    pool = pl.pallas_call(
        _pool_body,
        grid=(_NTC,),
        in_specs=[
            pl.BlockSpec((1, _D, 128, 128), lambda i: (i, 0, 0, 0)),
            pl.BlockSpec((1, 2, 128, 128), lambda i: (i, 0, 0, 0)),
        ],
        out_specs=[
            pl.BlockSpec((1, 2, _D), lambda i: (i, 0, 0)),
            pl.BlockSpec((1, 2, _D), lambda i: (i, 0, 0)),
            pl.BlockSpec((1, 2, _D), lambda i: (i, 0, 0)),
        ],
        out_shape=[
            jax.ShapeDtypeStruct((_NTC, 2, _D), jnp.float32),
            jax.ShapeDtypeStruct((_NTC, 2, _D), jnp.float32),
            jax.ShapeDtypeStruct((_NTC, 2, _D), jnp.float32),
        ],
    )
    t_sums, b_sums, cnts = pool(features, labels)

    ttc = t_sums.reshape(2 * _NTC, _D)
    btc = b_sums.reshape(2 * _NTC, _D)
    ctc = cnts[:, :, 0].reshape(2 * _NTC)

    finale = pl.pallas_call(
        _finale_mix_body,
        out_shape=jax.ShapeDtypeStruct((1, 1), jnp.float32),
    )
    loss = finale(
        ttc, btc, ctc.reshape(2 * _NTC, 1), ctc.reshape(1, 2 * _NTC),
        tsc, bsc, csc, csc.T,
        task_ids.reshape(_N2, 1), task_ids.reshape(1, _N2),
        perms,
    )
    return loss[0, 0]


_rng = np.random.default_rng(0)
_PERMS = np.stack(
    [_rng.permutation(_D)[:N_NEGATIVES] for _ in range(_N2)]).astype(np.int32)


def kernel(features, labels, tasks):
    task_ids = jnp.stack([2 * tasks, 2 * tasks + 1], axis=1).reshape(-1)
    return _run_mix(features, labels, task_ids.astype(jnp.int32), _PERMS)
